# Initial kernel scaffold; baseline (speedup 1.0000x reference)
import jax
import jax.numpy as jnp
from jax.experimental import pallas as pl


def kernel(x, senders, receivers, batch, num_graphs, W0, b0, W1, b1, W2, b2):
    raise NotImplementedError("write your pallas kernel here")


# speedup vs baseline: 8.4486x; 8.4486x over previous
import functools

import jax
import jax.numpy as jnp
from jax import lax
from jax.experimental import pallas as pl
from jax.experimental.pallas import tpu as pltpu
from jax.experimental.pallas import tpu_sc as plsc

N = 10000
E = 320000
D_IN = 128
H0 = 64
H1 = 128
D_OUT = 2
G = 100

NC = 2
NS = 16
NW = NC * NS
EPT = E // NW
CH = 80
NCH = EPT // CH
N_P = 10240
RPT = N_P // NS
ZB = 128

RB = 2048
NRB = N_P // RB

_f32 = jnp.float32


def _sc_mesh():
    return plsc.VectorSubcoreMesh(
        core_axis_name="c", subcore_axis_name="s", num_cores=NC, num_subcores=NS
    )


def _make_deg_kernel():
    DW = 16

    @functools.partial(
        pl.kernel,
        out_type=jax.ShapeDtypeStruct((NC, N_P, DW), _f32),
        mesh=_sc_mesh(),
        compiler_params=pltpu.CompilerParams(use_tc_tiling_on_sc=False),
        scratch_types=[
            pltpu.VMEM((NCH, CH), jnp.int32),
            pltpu.VMEM((NCH, CH), jnp.int32),
            pltpu.VMEM((CH, DW), _f32),
            pltpu.VMEM((CH, DW), _f32),
            pltpu.VMEM((ZB, DW), _f32),
            pltpu.VMEM_SHARED((N_P, DW), _f32),
        ],
    )
    def deg_kernel(send_hbm, recv_hbm, ones_s_hbm, ones_r_hbm, zeros_hbm,
                   deg_hbm, sidx_v, ridx_v, ones_s_v, ones_r_v, buf_v, acc):
        cid = lax.axis_index("c")
        sid = lax.axis_index("s")
        wid = cid * NS + sid
        r0 = sid * RPT
        pltpu.sync_copy(zeros_hbm, buf_v)
        for z in range(RPT // ZB):
            pltpu.sync_copy(buf_v, acc.at[pl.ds(r0 + z * ZB, ZB)])
        pltpu.sync_copy(ones_s_hbm, ones_s_v)
        pltpu.sync_copy(ones_r_hbm, ones_r_v)
        pltpu.sync_copy(send_hbm.at[wid], sidx_v)
        pltpu.sync_copy(recv_hbm.at[wid], ridx_v)
        plsc.subcore_barrier()

        def body(j, _):
            pltpu.sync_copy(ones_s_v, acc.at[sidx_v.at[j]], add=True)
            pltpu.sync_copy(ones_r_v, acc.at[ridx_v.at[j]], add=True)
            return _

        lax.fori_loop(0, NCH, body, None)
        plsc.subcore_barrier()
        for z in range(RPT // ZB):
            pltpu.sync_copy(acc.at[pl.ds(r0 + z * ZB, ZB)], buf_v)
            pltpu.sync_copy(buf_v, deg_hbm.at[cid, pl.ds(r0 + z * ZB, ZB)])

    return deg_kernel


def _make_prop_kernel(D):
    @functools.partial(
        pl.kernel,
        out_type=jax.ShapeDtypeStruct((NC, N_P, D), _f32),
        mesh=_sc_mesh(),
        compiler_params=pltpu.CompilerParams(use_tc_tiling_on_sc=False),
        scratch_types=[
            pltpu.VMEM((NCH, CH), jnp.int32),
            pltpu.VMEM((NCH, CH), jnp.int32),
            pltpu.VMEM((CH, D), _f32),
            pltpu.VMEM((ZB, D), _f32),
            pltpu.VMEM_SHARED((N_P, D), _f32),
            pltpu.SemaphoreType.DMA,
        ],
    )
    def prop_kernel(table_hbm, send_hbm, recv_hbm, zeros_hbm, out_hbm,
                    sidx_v, ridx_v, rows_v, buf_v, acc, sem):
        cid = lax.axis_index("c")
        sid = lax.axis_index("s")
        wid = cid * NS + sid
        r0 = sid * RPT
        pltpu.sync_copy(zeros_hbm, buf_v)
        for z in range(RPT // ZB):
            pltpu.sync_copy(buf_v, acc.at[pl.ds(r0 + z * ZB, ZB)])
        pltpu.sync_copy(send_hbm.at[wid], sidx_v)
        pltpu.sync_copy(recv_hbm.at[wid], ridx_v)
        plsc.subcore_barrier()

        def body(j, _):
            pltpu.async_copy(table_hbm.at[sidx_v.at[j]], rows_v, sem).wait()
            pltpu.sync_copy(rows_v, acc.at[ridx_v.at[j]], add=True)
            return _

        lax.fori_loop(0, NCH, body, None)
        plsc.subcore_barrier()
        for z in range(RPT // ZB):
            pltpu.sync_copy(acc.at[pl.ds(r0 + z * ZB, ZB)], buf_v)
            pltpu.sync_copy(buf_v, out_hbm.at[cid, pl.ds(r0 + z * ZB, ZB)])

    return prop_kernel


def _scales(d_ref):
    out_deg = d_ref[0, :, 0] + d_ref[1, :, 0]
    in_deg = d_ref[0, :, 8] + d_ref[1, :, 8]
    a = lax.rsqrt(jnp.maximum(out_deg, 1.0))
    c = lax.rsqrt(jnp.maximum(in_deg, 1.0))
    return a, c


def _tc1_body(x_ref, w0_ref, b0_ref, d_ref, t0_ref):
    a, _ = _scales(d_ref)
    h = jnp.dot(x_ref[...], w0_ref[...], preferred_element_type=_f32)
    h = h + b0_ref[0][None, :]
    t0_ref[...] = jnp.concatenate(
        [h * a[:, None], a[:, None], jnp.zeros((RB, CH - H0 - 1), _f32)], axis=1
    )


def _tc2_body(s0_ref, d_ref, t1_ref):
    a, c = _scales(d_ref)
    s0 = s0_ref[0] + s0_ref[1]
    h0 = c[:, None] * s0[:, :H0]
    t1_ref[...] = a[:, None] * jnp.maximum(h0, 0.0)


def _tc3_body(s1_ref, s0_ref, d_ref, w1_ref, b1_ref, w2_ref, t2_ref):
    a, c = _scales(d_ref)
    s1 = s1_ref[0] + s1_ref[1]
    t = s0_ref[0, :, H0] + s0_ref[1, :, H0]
    h1 = jnp.dot(c[:, None] * s1, w1_ref[...], preferred_element_type=_f32)
    h1 = h1 + (c * t)[:, None] * b1_ref[0][None, :]
    r1 = jnp.maximum(h1, 0.0)
    t2_ref[...] = jnp.dot(a[:, None] * r1, w2_ref[...], preferred_element_type=_f32)


def _tc4_body(s2_ref, s0_ref, d_ref, b2_ref, batch_ref, out_ref):
    i = pl.program_id(0)
    _, c = _scales(d_ref)
    s2 = s2_ref[0] + s2_ref[1]
    t = s0_ref[0, :, H0] + s0_ref[1, :, H0]
    h2 = c[:, None] * s2 + (c * t)[:, None] * b2_ref[0][None, :]
    bidx = batch_ref[0, 0, :]
    onehot = (bidx[:, None] == lax.broadcasted_iota(jnp.int32, (RB, 128), 1))
    contrib = lax.dot_general(
        onehot.astype(_f32), h2, (((0,), (0,)), ((), ())),
        preferred_element_type=_f32,
    )

    @pl.when(i == 0)
    def _():
        out_ref[...] = contrib

    @pl.when(i > 0)
    def _():
        out_ref[...] = out_ref[...] + contrib


def _deg_spec():
    return pl.BlockSpec((NC, RB, 16), lambda i: (0, i, 0))


def kernel(x, senders, receivers, batch, num_graphs, W0, b0, W1, b1, W2, b2):
    x = jnp.concatenate([x, jnp.zeros((N_P - N, D_IN), _f32)], axis=0)
    batch = jnp.concatenate([batch, jnp.zeros((N_P - N,), batch.dtype)], axis=0)
    send3 = senders.reshape(NW, NCH, CH)
    recv3 = receivers.reshape(NW, NCH, CH)
    b0_2d = jnp.broadcast_to(b0[None, :], (8, H0))
    b1_2d = jnp.broadcast_to(b1[None, :], (8, H1))
    b2_16 = jnp.zeros((8, 16), _f32).at[:, :D_OUT].set(b2[None, :])
    w2_16 = jnp.zeros((H1, 16), _f32).at[:, :D_OUT].set(W2)
    batch3 = batch.reshape(NRB, 1, RB)

    ones_s = jnp.zeros((CH, 16), _f32).at[:, :8].set(1.0)
    ones_r = jnp.zeros((CH, 16), _f32).at[:, 8:].set(1.0)
    zeros16 = jnp.zeros((ZB, 16), _f32)
    zeros64 = jnp.zeros((ZB, H0), _f32)
    zeros80 = jnp.zeros((ZB, CH), _f32)

    deg = _make_deg_kernel()(send3, recv3, ones_s, ones_r, zeros16)

    t0 = pl.pallas_call(
        _tc1_body,
        grid=(NRB,),
        in_specs=[
            pl.BlockSpec((RB, D_IN), lambda i: (i, 0)),
            pl.BlockSpec((D_IN, H0), lambda i: (0, 0)),
            pl.BlockSpec((8, H0), lambda i: (0, 0)),
            _deg_spec(),
        ],
        out_specs=pl.BlockSpec((RB, CH), lambda i: (i, 0)),
        out_shape=jax.ShapeDtypeStruct((N_P, CH), _f32),
    )(x, W0, b0_2d, deg)

    s0p = _make_prop_kernel(CH)(t0, send3, recv3, zeros80)

    t1 = pl.pallas_call(
        _tc2_body,
        grid=(NRB,),
        in_specs=[
            pl.BlockSpec((NC, RB, CH), lambda i: (0, i, 0)),
            _deg_spec(),
        ],
        out_specs=pl.BlockSpec((RB, H0), lambda i: (i, 0)),
        out_shape=jax.ShapeDtypeStruct((N_P, H0), _f32),
    )(s0p, deg)

    s1p = _make_prop_kernel(H0)(t1, send3, recv3, zeros64)

    t2 = pl.pallas_call(
        _tc3_body,
        grid=(NRB,),
        in_specs=[
            pl.BlockSpec((NC, RB, H0), lambda i: (0, i, 0)),
            pl.BlockSpec((NC, RB, CH), lambda i: (0, i, 0)),
            _deg_spec(),
            pl.BlockSpec((H0, H1), lambda i: (0, 0)),
            pl.BlockSpec((8, H1), lambda i: (0, 0)),
            pl.BlockSpec((H1, 16), lambda i: (0, 0)),
        ],
        out_specs=pl.BlockSpec((RB, 16), lambda i: (i, 0)),
        out_shape=jax.ShapeDtypeStruct((N_P, 16), _f32),
    )(s1p, s0p, deg, W1, b1_2d, w2_16)

    s2p = _make_prop_kernel(16)(t2, send3, recv3, zeros16)

    pooled = pl.pallas_call(
        _tc4_body,
        grid=(NRB,),
        in_specs=[
            pl.BlockSpec((NC, RB, 16), lambda i: (0, i, 0)),
            pl.BlockSpec((NC, RB, CH), lambda i: (0, i, 0)),
            _deg_spec(),
            pl.BlockSpec((8, 16), lambda i: (0, 0)),
            pl.BlockSpec((1, 1, RB), lambda i: (i, 0, 0)),
        ],
        out_specs=pl.BlockSpec((128, 16), lambda i: (0, 0)),
        out_shape=jax.ShapeDtypeStruct((128, 16), _f32),
    )(s2p, s0p, deg, b2_16, batch3)

    return pooled[:G, :D_OUT]


# speedup vs baseline: 15.8019x; 1.8704x over previous
import functools

import jax
import jax.numpy as jnp
from jax import lax
from jax.experimental import pallas as pl
from jax.experimental.pallas import tpu as pltpu
from jax.experimental.pallas import tpu_sc as plsc

N = 10000
E = 320000
D_IN = 128
H0 = 64
H1 = 128
D_OUT = 2
G = 100

NC = 2
NS = 16
NW = NC * NS
EPT = E // NW
CH = 125
NCH = EPT // CH
NB = 4
NGRP = NCH // NB
N_P = 10240
RPT = N_P // NS
ZB = 128

TW = 80

RB = 2048
NRB = N_P // RB

_f32 = jnp.float32


def _sc_mesh():
    return plsc.VectorSubcoreMesh(
        core_axis_name="c", subcore_axis_name="s", num_cores=NC, num_subcores=NS
    )


def _make_deg_kernel():
    DW = 16

    @functools.partial(
        pl.kernel,
        out_type=jax.ShapeDtypeStruct((NC, N_P, DW), _f32),
        mesh=_sc_mesh(),
        compiler_params=pltpu.CompilerParams(use_tc_tiling_on_sc=False),
        scratch_types=[
            pltpu.VMEM((NCH, CH), jnp.int32),
            pltpu.VMEM((NCH, CH), jnp.int32),
            pltpu.VMEM((CH, DW), _f32),
            pltpu.VMEM((CH, DW), _f32),
            pltpu.VMEM((ZB, DW), _f32),
            pltpu.VMEM_SHARED((N_P, DW), _f32),
        ],
    )
    def deg_kernel(send_hbm, recv_hbm, ones_s_hbm, ones_r_hbm, zeros_hbm,
                   deg_hbm, sidx_v, ridx_v, ones_s_v, ones_r_v, buf_v, acc):
        cid = lax.axis_index("c")
        sid = lax.axis_index("s")
        wid = cid * NS + sid
        r0 = sid * RPT
        pltpu.sync_copy(zeros_hbm, buf_v)
        for z in range(RPT // ZB):
            pltpu.sync_copy(buf_v, acc.at[pl.ds(r0 + z * ZB, ZB)])
        pltpu.sync_copy(ones_s_hbm, ones_s_v)
        pltpu.sync_copy(ones_r_hbm, ones_r_v)
        pltpu.sync_copy(send_hbm.at[wid], sidx_v)
        pltpu.sync_copy(recv_hbm.at[wid], ridx_v)
        plsc.subcore_barrier()

        def body(j, _):
            pltpu.sync_copy(ones_s_v, acc.at[sidx_v.at[j]], add=True)
            pltpu.sync_copy(ones_r_v, acc.at[ridx_v.at[j]], add=True)
            return _

        lax.fori_loop(0, NCH, body, None)
        plsc.subcore_barrier()
        for z in range(RPT // ZB):
            pltpu.sync_copy(acc.at[pl.ds(r0 + z * ZB, ZB)], buf_v)
            pltpu.sync_copy(buf_v, deg_hbm.at[cid, pl.ds(r0 + z * ZB, ZB)])

    return deg_kernel


def _make_prop_kernel(D):
    @functools.partial(
        pl.kernel,
        out_type=jax.ShapeDtypeStruct((NC, N_P, D), _f32),
        mesh=_sc_mesh(),
        compiler_params=pltpu.CompilerParams(use_tc_tiling_on_sc=False),
        scratch_types=[
            pltpu.VMEM((NCH, CH), jnp.int32),
            pltpu.VMEM((NCH, CH), jnp.int32),
            [pltpu.VMEM((CH, D), _f32) for _ in range(NB)],
            pltpu.VMEM((ZB, D), _f32),
            pltpu.VMEM_SHARED((N_P, D), _f32),
            [pltpu.SemaphoreType.DMA for _ in range(NB)],
        ],
    )
    def prop_kernel(table_hbm, send_hbm, recv_hbm, zeros_hbm, out_hbm,
                    sidx_v, ridx_v, rows, buf_v, acc, sems):
        cid = lax.axis_index("c")
        sid = lax.axis_index("s")
        wid = cid * NS + sid
        r0 = sid * RPT
        pltpu.sync_copy(zeros_hbm, buf_v)
        for z in range(RPT // ZB):
            pltpu.sync_copy(buf_v, acc.at[pl.ds(r0 + z * ZB, ZB)])
        pltpu.sync_copy(send_hbm.at[wid], sidx_v)
        pltpu.sync_copy(recv_hbm.at[wid], ridx_v)
        plsc.subcore_barrier()

        for b in range(NB):
            pltpu.async_copy(table_hbm.at[sidx_v.at[b]], rows[b], sems[b])

        def grp(g, _):
            for b in range(NB):
                j = g * NB + b
                pltpu.make_async_copy(
                    table_hbm.at[sidx_v.at[j]], rows[b], sems[b]).wait()
                pltpu.sync_copy(rows[b], acc.at[ridx_v.at[j]], add=True)
                pltpu.async_copy(
                    table_hbm.at[sidx_v.at[j + NB]], rows[b], sems[b])
            return _

        lax.fori_loop(0, NGRP - 1, grp, None)
        for b in range(NB):
            j = (NGRP - 1) * NB + b
            pltpu.make_async_copy(
                table_hbm.at[sidx_v.at[j]], rows[b], sems[b]).wait()
            pltpu.sync_copy(rows[b], acc.at[ridx_v.at[j]], add=True)
        plsc.subcore_barrier()
        for z in range(RPT // ZB):
            pltpu.sync_copy(acc.at[pl.ds(r0 + z * ZB, ZB)], buf_v)
            pltpu.sync_copy(buf_v, out_hbm.at[cid, pl.ds(r0 + z * ZB, ZB)])

    return prop_kernel


def _scales(d_ref):
    out_deg = d_ref[0, :, 0] + d_ref[1, :, 0]
    in_deg = d_ref[0, :, 8] + d_ref[1, :, 8]
    a = lax.rsqrt(jnp.maximum(out_deg, 1.0))
    c = lax.rsqrt(jnp.maximum(in_deg, 1.0))
    return a, c


def _tc1_body(x_ref, w0_ref, b0_ref, d_ref, t0_ref):
    a, _ = _scales(d_ref)
    h = jnp.dot(x_ref[...], w0_ref[...], preferred_element_type=_f32)
    h = h + b0_ref[0][None, :]
    t0_ref[...] = jnp.concatenate(
        [h * a[:, None], a[:, None], jnp.zeros((RB, TW - H0 - 1), _f32)], axis=1
    )


def _tc2_body(s0_ref, d_ref, t1_ref):
    a, c = _scales(d_ref)
    s0 = s0_ref[0] + s0_ref[1]
    h0 = c[:, None] * s0[:, :H0]
    t1_ref[...] = a[:, None] * jnp.maximum(h0, 0.0)


def _tc3_body(s1_ref, s0_ref, d_ref, w1_ref, b1_ref, w2_ref, t2_ref):
    a, c = _scales(d_ref)
    s1 = s1_ref[0] + s1_ref[1]
    t = s0_ref[0, :, H0] + s0_ref[1, :, H0]
    h1 = jnp.dot(c[:, None] * s1, w1_ref[...], preferred_element_type=_f32)
    h1 = h1 + (c * t)[:, None] * b1_ref[0][None, :]
    r1 = jnp.maximum(h1, 0.0)
    t2_ref[...] = jnp.dot(a[:, None] * r1, w2_ref[...], preferred_element_type=_f32)


def _tc4_body(s2_ref, s0_ref, d_ref, b2_ref, batch_ref, out_ref):
    i = pl.program_id(0)
    _, c = _scales(d_ref)
    s2 = s2_ref[0] + s2_ref[1]
    t = s0_ref[0, :, H0] + s0_ref[1, :, H0]
    h2 = c[:, None] * s2 + (c * t)[:, None] * b2_ref[0][None, :]
    bidx = batch_ref[0, 0, :]
    onehot = (bidx[:, None] == lax.broadcasted_iota(jnp.int32, (RB, 128), 1))
    contrib = lax.dot_general(
        onehot.astype(_f32), h2, (((0,), (0,)), ((), ())),
        preferred_element_type=_f32,
    )

    @pl.when(i == 0)
    def _():
        out_ref[...] = contrib

    @pl.when(i > 0)
    def _():
        out_ref[...] = out_ref[...] + contrib


def _deg_spec():
    return pl.BlockSpec((NC, RB, 16), lambda i: (0, i, 0))


def kernel(x, senders, receivers, batch, num_graphs, W0, b0, W1, b1, W2, b2):
    x = jnp.concatenate([x, jnp.zeros((N_P - N, D_IN), _f32)], axis=0)
    batch = jnp.concatenate([batch, jnp.zeros((N_P - N,), batch.dtype)], axis=0)
    send3 = senders.reshape(NW, NCH, CH)
    recv3 = receivers.reshape(NW, NCH, CH)
    b0_2d = jnp.broadcast_to(b0[None, :], (8, H0))
    b1_2d = jnp.broadcast_to(b1[None, :], (8, H1))
    b2_16 = jnp.zeros((8, 16), _f32).at[:, :D_OUT].set(b2[None, :])
    w2_16 = jnp.zeros((H1, 16), _f32).at[:, :D_OUT].set(W2)
    batch3 = batch.reshape(NRB, 1, RB)

    ones_s = jnp.zeros((CH, 16), _f32).at[:, :8].set(1.0)
    ones_r = jnp.zeros((CH, 16), _f32).at[:, 8:].set(1.0)
    zeros16 = jnp.zeros((ZB, 16), _f32)
    zeros64 = jnp.zeros((ZB, H0), _f32)
    zeros80 = jnp.zeros((ZB, TW), _f32)

    deg = _make_deg_kernel()(send3, recv3, ones_s, ones_r, zeros16)

    t0 = pl.pallas_call(
        _tc1_body,
        grid=(NRB,),
        in_specs=[
            pl.BlockSpec((RB, D_IN), lambda i: (i, 0)),
            pl.BlockSpec((D_IN, H0), lambda i: (0, 0)),
            pl.BlockSpec((8, H0), lambda i: (0, 0)),
            _deg_spec(),
        ],
        out_specs=pl.BlockSpec((RB, TW), lambda i: (i, 0)),
        out_shape=jax.ShapeDtypeStruct((N_P, TW), _f32),
    )(x, W0, b0_2d, deg)

    s0p = _make_prop_kernel(TW)(t0, send3, recv3, zeros80)

    t1 = pl.pallas_call(
        _tc2_body,
        grid=(NRB,),
        in_specs=[
            pl.BlockSpec((NC, RB, TW), lambda i: (0, i, 0)),
            _deg_spec(),
        ],
        out_specs=pl.BlockSpec((RB, H0), lambda i: (i, 0)),
        out_shape=jax.ShapeDtypeStruct((N_P, H0), _f32),
    )(s0p, deg)

    s1p = _make_prop_kernel(H0)(t1, send3, recv3, zeros64)

    t2 = pl.pallas_call(
        _tc3_body,
        grid=(NRB,),
        in_specs=[
            pl.BlockSpec((NC, RB, H0), lambda i: (0, i, 0)),
            pl.BlockSpec((NC, RB, TW), lambda i: (0, i, 0)),
            _deg_spec(),
            pl.BlockSpec((H0, H1), lambda i: (0, 0)),
            pl.BlockSpec((8, H1), lambda i: (0, 0)),
            pl.BlockSpec((H1, 16), lambda i: (0, 0)),
        ],
        out_specs=pl.BlockSpec((RB, 16), lambda i: (i, 0)),
        out_shape=jax.ShapeDtypeStruct((N_P, 16), _f32),
    )(s1p, s0p, deg, W1, b1_2d, w2_16)

    s2p = _make_prop_kernel(16)(t2, send3, recv3, zeros16)

    pooled = pl.pallas_call(
        _tc4_body,
        grid=(NRB,),
        in_specs=[
            pl.BlockSpec((NC, RB, 16), lambda i: (0, i, 0)),
            pl.BlockSpec((NC, RB, TW), lambda i: (0, i, 0)),
            _deg_spec(),
            pl.BlockSpec((8, 16), lambda i: (0, 0)),
            pl.BlockSpec((1, 1, RB), lambda i: (i, 0, 0)),
        ],
        out_specs=pl.BlockSpec((128, 16), lambda i: (0, 0)),
        out_shape=jax.ShapeDtypeStruct((128, 16), _f32),
    )(s2p, s0p, deg, b2_16, batch3)

    return pooled[:G, :D_OUT]


# speedup vs baseline: 16.2122x; 1.0260x over previous
import functools

import jax
import jax.numpy as jnp
from jax import lax
from jax.experimental import pallas as pl
from jax.experimental.pallas import tpu as pltpu
from jax.experimental.pallas import tpu_sc as plsc

N = 10000
E = 320000
D_IN = 128
H0 = 64
H1 = 128
D_OUT = 2
G = 100

NC = 2
NS = 16
NW = NC * NS
EPT = E // NW
CH = 125
NCH = EPT // CH
NB = 4
NGRP = NCH // NB
N_P = 10240
RPT = N_P // NS
ZB = 128

TW = 72
DW = 8

RB = 2000
NRB = N // RB

_f32 = jnp.float32


def _sc_mesh():
    return plsc.VectorSubcoreMesh(
        core_axis_name="c", subcore_axis_name="s", num_cores=NC, num_subcores=NS
    )


def _make_deg_kernel():
    @functools.partial(
        pl.kernel,
        out_type=jax.ShapeDtypeStruct((NC, N_P, DW), _f32),
        mesh=_sc_mesh(),
        compiler_params=pltpu.CompilerParams(use_tc_tiling_on_sc=False),
        scratch_types=[
            pltpu.VMEM((NCH, CH), jnp.int32),
            pltpu.VMEM((NCH, CH), jnp.int32),
            pltpu.VMEM((CH, DW), _f32),
            pltpu.VMEM((CH, DW), _f32),
            pltpu.VMEM((ZB, DW), _f32),
            pltpu.VMEM_SHARED((N_P, DW), _f32),
        ],
    )
    def deg_kernel(send_hbm, recv_hbm, ones_s_hbm, ones_r_hbm, zeros_hbm,
                   deg_hbm, sidx_v, ridx_v, ones_s_v, ones_r_v, buf_v, acc):
        cid = lax.axis_index("c")
        sid = lax.axis_index("s")
        wid = cid * NS + sid
        r0 = sid * RPT
        pltpu.sync_copy(zeros_hbm, buf_v)
        for z in range(RPT // ZB):
            pltpu.sync_copy(buf_v, acc.at[pl.ds(r0 + z * ZB, ZB)])
        pltpu.sync_copy(ones_s_hbm, ones_s_v)
        pltpu.sync_copy(ones_r_hbm, ones_r_v)
        pltpu.sync_copy(send_hbm.at[wid], sidx_v)
        pltpu.sync_copy(recv_hbm.at[wid], ridx_v)
        plsc.subcore_barrier()

        def body(j, _):
            pltpu.sync_copy(ones_s_v, acc.at[sidx_v.at[j]], add=True)
            pltpu.sync_copy(ones_r_v, acc.at[ridx_v.at[j]], add=True)
            return _

        lax.fori_loop(0, NCH, body, None)
        plsc.subcore_barrier()
        for z in range(RPT // ZB):
            pltpu.sync_copy(acc.at[pl.ds(r0 + z * ZB, ZB)], buf_v)
            pltpu.sync_copy(buf_v, deg_hbm.at[cid, pl.ds(r0 + z * ZB, ZB)])

    return deg_kernel


def _make_prop_kernel(D):
    @functools.partial(
        pl.kernel,
        out_type=jax.ShapeDtypeStruct((NC, N_P, D), _f32),
        mesh=_sc_mesh(),
        compiler_params=pltpu.CompilerParams(use_tc_tiling_on_sc=False),
        scratch_types=[
            pltpu.VMEM((NCH, CH), jnp.int32),
            pltpu.VMEM((NCH, CH), jnp.int32),
            [pltpu.VMEM((CH, D), _f32) for _ in range(NB)],
            pltpu.VMEM((ZB, D), _f32),
            pltpu.VMEM_SHARED((N_P, D), _f32),
            [pltpu.SemaphoreType.DMA for _ in range(NB)],
        ],
    )
    def prop_kernel(table_hbm, send_hbm, recv_hbm, zeros_hbm, out_hbm,
                    sidx_v, ridx_v, rows, buf_v, acc, sems):
        cid = lax.axis_index("c")
        sid = lax.axis_index("s")
        wid = cid * NS + sid
        r0 = sid * RPT
        pltpu.sync_copy(zeros_hbm, buf_v)
        for z in range(RPT // ZB):
            pltpu.sync_copy(buf_v, acc.at[pl.ds(r0 + z * ZB, ZB)])
        pltpu.sync_copy(send_hbm.at[wid], sidx_v)
        pltpu.sync_copy(recv_hbm.at[wid], ridx_v)
        plsc.subcore_barrier()

        for b in range(NB):
            pltpu.async_copy(table_hbm.at[sidx_v.at[b]], rows[b], sems[b])

        def grp(g, _):
            for b in range(NB):
                j = g * NB + b
                pltpu.make_async_copy(
                    table_hbm.at[sidx_v.at[j]], rows[b], sems[b]).wait()
                pltpu.sync_copy(rows[b], acc.at[ridx_v.at[j]], add=True)
                pltpu.async_copy(
                    table_hbm.at[sidx_v.at[j + NB]], rows[b], sems[b])
            return _

        lax.fori_loop(0, NGRP - 1, grp, None)
        for b in range(NB):
            j = (NGRP - 1) * NB + b
            pltpu.make_async_copy(
                table_hbm.at[sidx_v.at[j]], rows[b], sems[b]).wait()
            pltpu.sync_copy(rows[b], acc.at[ridx_v.at[j]], add=True)
        plsc.subcore_barrier()
        for z in range(RPT // ZB):
            pltpu.sync_copy(acc.at[pl.ds(r0 + z * ZB, ZB)], buf_v)
            pltpu.sync_copy(buf_v, out_hbm.at[cid, pl.ds(r0 + z * ZB, ZB)])

    return prop_kernel


def _scales(d_ref):
    out_deg = d_ref[0, :, 0] + d_ref[1, :, 0]
    in_deg = d_ref[0, :, 4] + d_ref[1, :, 4]
    a = lax.rsqrt(jnp.maximum(out_deg, 1.0))
    c = lax.rsqrt(jnp.maximum(in_deg, 1.0))
    return a, c


def _tc1a_body(x_ref, w0_ref, b0_ref, h_ref):
    h = jnp.dot(x_ref[...], w0_ref[...], preferred_element_type=_f32)
    h_ref[...] = h + b0_ref[0][None, :]


def _tc1b_body(h_ref, d_ref, t0_ref):
    a, _ = _scales(d_ref)
    h = h_ref[...]
    t0_ref[...] = jnp.concatenate(
        [h * a[:, None], a[:, None], jnp.zeros((RB, TW - H0 - 1), _f32)], axis=1
    )


def _tc2_body(s0_ref, d_ref, t1_ref):
    a, c = _scales(d_ref)
    s0 = s0_ref[0] + s0_ref[1]
    h0 = c[:, None] * s0[:, :H0]
    t1_ref[...] = a[:, None] * jnp.maximum(h0, 0.0)


def _tc3_body(s1_ref, s0_ref, d_ref, w1_ref, b1_ref, w2_ref, t2_ref):
    a, c = _scales(d_ref)
    s1 = s1_ref[0] + s1_ref[1]
    t = s0_ref[0, :, H0] + s0_ref[1, :, H0]
    h1 = jnp.dot(c[:, None] * s1, w1_ref[...], preferred_element_type=_f32)
    h1 = h1 + (c * t)[:, None] * b1_ref[0][None, :]
    r1 = jnp.maximum(h1, 0.0)
    t2_ref[...] = jnp.dot(a[:, None] * r1, w2_ref[...], preferred_element_type=_f32)


def _tc4_body(s2_ref, s0_ref, d_ref, b2_ref, batch_ref, out_ref):
    i = pl.program_id(0)
    _, c = _scales(d_ref)
    s2 = s2_ref[0] + s2_ref[1]
    t = s0_ref[0, :, H0] + s0_ref[1, :, H0]
    h2 = c[:, None] * s2 + (c * t)[:, None] * b2_ref[0][None, :]
    bidx = batch_ref[0, 0, :]
    onehot = (bidx[:, None] == lax.broadcasted_iota(jnp.int32, (RB, 128), 1))
    contrib = lax.dot_general(
        onehot.astype(_f32), h2, (((0,), (0,)), ((), ())),
        preferred_element_type=_f32,
    )

    @pl.when(i == 0)
    def _():
        out_ref[...] = contrib

    @pl.when(i > 0)
    def _():
        out_ref[...] = out_ref[...] + contrib


def _deg_spec():
    return pl.BlockSpec((NC, RB, DW), lambda i: (0, i, 0))


def kernel(x, senders, receivers, batch, num_graphs, W0, b0, W1, b1, W2, b2):
    send3 = senders.reshape(NW, NCH, CH)
    recv3 = receivers.reshape(NW, NCH, CH)
    b0_2d = jnp.broadcast_to(b0[None, :], (8, H0))
    b1_2d = jnp.broadcast_to(b1[None, :], (8, H1))
    b2_16 = jnp.zeros((8, 16), _f32).at[:, :D_OUT].set(b2[None, :])
    w2_16 = jnp.zeros((H1, 16), _f32).at[:, :D_OUT].set(W2)
    batch3 = batch.reshape(NRB, 1, RB)

    ones_s = jnp.zeros((CH, DW), _f32).at[:, :4].set(1.0)
    ones_r = jnp.zeros((CH, DW), _f32).at[:, 4:].set(1.0)
    zeros8 = jnp.zeros((ZB, DW), _f32)
    zeros16 = jnp.zeros((ZB, 16), _f32)
    zeros64 = jnp.zeros((ZB, H0), _f32)
    zeros80 = jnp.zeros((ZB, TW), _f32)

    deg = _make_deg_kernel()(send3, recv3, ones_s, ones_r, zeros8)

    h0raw = pl.pallas_call(
        _tc1a_body,
        grid=(NRB,),
        in_specs=[
            pl.BlockSpec((RB, D_IN), lambda i: (i, 0)),
            pl.BlockSpec((D_IN, H0), lambda i: (0, 0)),
            pl.BlockSpec((8, H0), lambda i: (0, 0)),
        ],
        out_specs=pl.BlockSpec((RB, H0), lambda i: (i, 0)),
        out_shape=jax.ShapeDtypeStruct((N, H0), _f32),
    )(x, W0, b0_2d)

    t0 = pl.pallas_call(
        _tc1b_body,
        grid=(NRB,),
        in_specs=[
            pl.BlockSpec((RB, H0), lambda i: (i, 0)),
            _deg_spec(),
        ],
        out_specs=pl.BlockSpec((RB, TW), lambda i: (i, 0)),
        out_shape=jax.ShapeDtypeStruct((N_P, TW), _f32),
    )(h0raw, deg)

    s0p = _make_prop_kernel(TW)(t0, send3, recv3, zeros80)

    t1 = pl.pallas_call(
        _tc2_body,
        grid=(NRB,),
        in_specs=[
            pl.BlockSpec((NC, RB, TW), lambda i: (0, i, 0)),
            _deg_spec(),
        ],
        out_specs=pl.BlockSpec((RB, H0), lambda i: (i, 0)),
        out_shape=jax.ShapeDtypeStruct((N_P, H0), _f32),
    )(s0p, deg)

    s1p = _make_prop_kernel(H0)(t1, send3, recv3, zeros64)

    t2 = pl.pallas_call(
        _tc3_body,
        grid=(NRB,),
        in_specs=[
            pl.BlockSpec((NC, RB, H0), lambda i: (0, i, 0)),
            pl.BlockSpec((NC, RB, TW), lambda i: (0, i, 0)),
            _deg_spec(),
            pl.BlockSpec((H0, H1), lambda i: (0, 0)),
            pl.BlockSpec((8, H1), lambda i: (0, 0)),
            pl.BlockSpec((H1, 16), lambda i: (0, 0)),
        ],
        out_specs=pl.BlockSpec((RB, 16), lambda i: (i, 0)),
        out_shape=jax.ShapeDtypeStruct((N_P, 16), _f32),
    )(s1p, s0p, deg, W1, b1_2d, w2_16)

    s2p = _make_prop_kernel(16)(t2, send3, recv3, zeros16)

    pooled = pl.pallas_call(
        _tc4_body,
        grid=(NRB,),
        in_specs=[
            pl.BlockSpec((NC, RB, 16), lambda i: (0, i, 0)),
            pl.BlockSpec((NC, RB, TW), lambda i: (0, i, 0)),
            _deg_spec(),
            pl.BlockSpec((8, 16), lambda i: (0, 0)),
            pl.BlockSpec((1, 1, RB), lambda i: (i, 0, 0)),
        ],
        out_specs=pl.BlockSpec((128, 16), lambda i: (0, 0)),
        out_shape=jax.ShapeDtypeStruct((128, 16), _f32),
    )(s2p, s0p, deg, b2_16, batch3)

    return pooled[:G, :D_OUT]


# speedup vs baseline: 17.0145x; 1.0495x over previous
import functools

import jax
import jax.numpy as jnp
from jax import lax
from jax.experimental import pallas as pl
from jax.experimental.pallas import tpu as pltpu
from jax.experimental.pallas import tpu_sc as plsc

N = 10000
E = 320000
D_IN = 128
H0 = 64
H1 = 128
D_OUT = 2
G = 100

NC = 2
NS = 16
NW = NC * NS
EPT = E // NW
CH = 125
NCH = EPT // CH
NB = 4
NGRP = NCH // NB
N_P = 10240
RPT = N_P // NS
ZB = 128

TW = 72
DW = 8

RB = 2000
NRB = N // RB

_f32 = jnp.float32


def _sc_mesh():
    return plsc.VectorSubcoreMesh(
        core_axis_name="c", subcore_axis_name="s", num_cores=NC, num_subcores=NS
    )


def _make_deg_kernel():
    @functools.partial(
        pl.kernel,
        out_type=jax.ShapeDtypeStruct((NC, N_P, DW), _f32),
        mesh=_sc_mesh(),
        compiler_params=pltpu.CompilerParams(use_tc_tiling_on_sc=False),
        scratch_types=[
            pltpu.VMEM((NCH, CH), jnp.int32),
            pltpu.VMEM((NCH, CH), jnp.int32),
            pltpu.VMEM((CH, DW), _f32),
            pltpu.VMEM((CH, DW), _f32),
            pltpu.VMEM((ZB, DW), _f32),
            pltpu.VMEM_SHARED((N_P, DW), _f32),
        ],
    )
    def deg_kernel(send_hbm, recv_hbm, ones_s_hbm, ones_r_hbm, zeros_hbm,
                   deg_hbm, sidx_v, ridx_v, ones_s_v, ones_r_v, buf_v, acc):
        cid = lax.axis_index("c")
        sid = lax.axis_index("s")
        wid = cid * NS + sid
        r0 = sid * RPT
        pltpu.sync_copy(zeros_hbm, buf_v)
        for z in range(RPT // ZB):
            pltpu.sync_copy(buf_v, acc.at[pl.ds(r0 + z * ZB, ZB)])
        pltpu.sync_copy(ones_s_hbm, ones_s_v)
        pltpu.sync_copy(ones_r_hbm, ones_r_v)
        pltpu.sync_copy(send_hbm.at[wid], sidx_v)
        pltpu.sync_copy(recv_hbm.at[wid], ridx_v)
        plsc.subcore_barrier()

        def body(j, _):
            pltpu.sync_copy(ones_s_v, acc.at[sidx_v.at[j]], add=True)
            pltpu.sync_copy(ones_r_v, acc.at[ridx_v.at[j]], add=True)
            return _

        lax.fori_loop(0, NCH, body, None)
        plsc.subcore_barrier()
        for z in range(RPT // ZB):
            pltpu.sync_copy(acc.at[pl.ds(r0 + z * ZB, ZB)], buf_v)
            pltpu.sync_copy(buf_v, deg_hbm.at[cid, pl.ds(r0 + z * ZB, ZB)])

    return deg_kernel


def _make_prop_kernel(D):
    @functools.partial(
        pl.kernel,
        out_type=jax.ShapeDtypeStruct((NC, N_P, D), _f32),
        mesh=_sc_mesh(),
        compiler_params=pltpu.CompilerParams(use_tc_tiling_on_sc=False),
        scratch_types=[
            pltpu.VMEM((NCH, CH), jnp.int32),
            pltpu.VMEM((NCH, CH), jnp.int32),
            [pltpu.VMEM((CH, D), _f32) for _ in range(NB)],
            pltpu.VMEM((ZB, D), _f32),
            pltpu.VMEM_SHARED((N_P, D), _f32),
            [pltpu.SemaphoreType.DMA for _ in range(NB)],
        ],
    )
    def prop_kernel(table_hbm, send_hbm, recv_hbm, zeros_hbm, out_hbm,
                    sidx_v, ridx_v, rows, buf_v, acc, sems):
        cid = lax.axis_index("c")
        sid = lax.axis_index("s")
        wid = cid * NS + sid
        r0 = sid * RPT
        pltpu.sync_copy(zeros_hbm, buf_v)
        for z in range(RPT // ZB):
            pltpu.sync_copy(buf_v, acc.at[pl.ds(r0 + z * ZB, ZB)])
        pltpu.sync_copy(send_hbm.at[wid], sidx_v)
        pltpu.sync_copy(recv_hbm.at[wid], ridx_v)
        plsc.subcore_barrier()

        for b in range(NB):
            pltpu.async_copy(table_hbm.at[sidx_v.at[b]], rows[b], sems[b])

        def grp(g, _):
            for b in range(NB):
                j = g * NB + b
                pltpu.make_async_copy(
                    table_hbm.at[sidx_v.at[j]], rows[b], sems[b]).wait()
                pltpu.sync_copy(rows[b], acc.at[ridx_v.at[j]], add=True)
                pltpu.async_copy(
                    table_hbm.at[sidx_v.at[j + NB]], rows[b], sems[b])
            return _

        lax.fori_loop(0, NGRP - 1, grp, None)
        for b in range(NB):
            j = (NGRP - 1) * NB + b
            pltpu.make_async_copy(
                table_hbm.at[sidx_v.at[j]], rows[b], sems[b]).wait()
            pltpu.sync_copy(rows[b], acc.at[ridx_v.at[j]], add=True)
        plsc.subcore_barrier()
        for z in range(RPT // ZB):
            pltpu.sync_copy(acc.at[pl.ds(r0 + z * ZB, ZB)], buf_v)
            pltpu.sync_copy(buf_v, out_hbm.at[cid, pl.ds(r0 + z * ZB, ZB)])

    return prop_kernel


def _scales(d_ref):
    out_deg = d_ref[0, :, 0] + d_ref[1, :, 0]
    in_deg = d_ref[0, :, 4] + d_ref[1, :, 4]
    a = lax.rsqrt(jnp.maximum(out_deg, 1.0))
    c = lax.rsqrt(jnp.maximum(in_deg, 1.0))
    return a, c


def _tc1a_body(x_ref, w0_ref, b0_ref, h_ref):
    h = jnp.dot(x_ref[...], w0_ref[...], preferred_element_type=_f32)
    h_ref[...] = h + b0_ref[0][None, :]


def _tc1b_body(h_ref, d_ref, t0_ref):
    a, _ = _scales(d_ref)
    h = h_ref[...]
    t0_ref[...] = jnp.concatenate(
        [h * a[:, None], a[:, None], jnp.zeros((RB, TW - H0 - 1), _f32)], axis=1
    )


def _tc2_body(s0_ref, d_ref, b2_ref, t1_ref, aux_ref):
    a, c = _scales(d_ref)
    s0 = s0_ref[0] + s0_ref[1]
    h0 = c[:, None] * s0[:, :H0]
    t1_ref[...] = a[:, None] * jnp.maximum(h0, 0.0)
    t = s0_ref[0, :, H0] + s0_ref[1, :, H0]
    ct = c * t
    b2m = b2_ref[...]
    aux_ref[...] = jnp.concatenate(
        [a[None, :], c[None, :], ct[None, :],
         (ct * b2m[0, 0])[None, :], (ct * b2m[0, 1])[None, :],
         jnp.zeros((3, N_P), _f32)], axis=0)


def _tc3_body(s1_ref, aux_ref, w1_ref, b1_ref, w2_ref, t2_ref):
    s1 = s1_ref[0] + s1_ref[1]
    a, c, ct = aux_ref[0], aux_ref[1], aux_ref[2]
    h1 = jnp.dot(c[:, None] * s1, w1_ref[...], preferred_element_type=_f32)
    h1 = h1 + ct[:, None] * b1_ref[0][None, :]
    r1 = jnp.maximum(h1, 0.0)
    t2_ref[...] = jnp.dot(a[:, None] * r1, w2_ref[...], preferred_element_type=_f32)


def _make_prop_pool_kernel():
    D = 16

    @functools.partial(
        pl.kernel,
        out_type=jax.ShapeDtypeStruct((NC, 128, 16), _f32),
        mesh=_sc_mesh(),
        compiler_params=pltpu.CompilerParams(
            use_tc_tiling_on_sc=False, needs_layout_passes=False),
        scratch_types=[
            pltpu.VMEM((NCH, CH), jnp.int32),
            pltpu.VMEM((NCH, CH), jnp.int32),
            [pltpu.VMEM((CH, D), _f32) for _ in range(NB)],
            pltpu.VMEM((ZB, D), _f32),
            pltpu.VMEM((RPT, D), _f32),
            pltpu.VMEM((RPT,), _f32),
            pltpu.VMEM((RPT,), _f32),
            pltpu.VMEM((RPT,), _f32),
            pltpu.VMEM((RPT,), jnp.int32),
            pltpu.VMEM((128,), jnp.int32),
            pltpu.VMEM((128, 16), _f32),
            pltpu.VMEM_SHARED((N_P, D), _f32),
            pltpu.VMEM_SHARED((128, 16), _f32),
            [pltpu.SemaphoreType.DMA for _ in range(NB)],
        ],
    )
    def prop_pool(table_hbm, send_hbm, recv_hbm, zeros_hbm,
                  aux_hbm, batch_hbm, iota_hbm, out_hbm,
                  sidx_v, ridx_v, rows, buf_v, s2_v, c_v, p0_v, p1_v, b_v,
                  iota_v, pool_v, acc, pool_s, sems):
        cid = lax.axis_index("c")
        sid = lax.axis_index("s")
        wid = cid * NS + sid
        r0 = sid * RPT
        pltpu.sync_copy(zeros_hbm, buf_v)
        for z in range(RPT // ZB):
            pltpu.sync_copy(buf_v, acc.at[pl.ds(r0 + z * ZB, ZB)])
        pltpu.sync_copy(zeros_hbm, pool_v)

        @pl.when(sid == 0)
        def _():
            pltpu.sync_copy(pool_v, pool_s)

        pltpu.sync_copy(aux_hbm.at[1, pl.ds(r0, RPT)], c_v)
        pltpu.sync_copy(aux_hbm.at[3, pl.ds(r0, RPT)], p0_v)
        pltpu.sync_copy(aux_hbm.at[4, pl.ds(r0, RPT)], p1_v)
        pltpu.sync_copy(batch_hbm.at[pl.ds(r0, RPT)], b_v)
        pltpu.sync_copy(iota_hbm, iota_v)
        pltpu.sync_copy(send_hbm.at[wid], sidx_v)
        pltpu.sync_copy(recv_hbm.at[wid], ridx_v)
        plsc.subcore_barrier()

        for b in range(NB):
            pltpu.async_copy(table_hbm.at[sidx_v.at[b]], rows[b], sems[b])

        def grp(g, _):
            for b in range(NB):
                j = g * NB + b
                pltpu.make_async_copy(
                    table_hbm.at[sidx_v.at[j]], rows[b], sems[b]).wait()
                pltpu.sync_copy(rows[b], acc.at[ridx_v.at[j]], add=True)
                pltpu.async_copy(
                    table_hbm.at[sidx_v.at[j + NB]], rows[b], sems[b])
            return _

        lax.fori_loop(0, NGRP - 1, grp, None)
        for b in range(NB):
            j = (NGRP - 1) * NB + b
            pltpu.make_async_copy(
                table_hbm.at[sidx_v.at[j]], rows[b], sems[b]).wait()
            pltpu.sync_copy(rows[b], acc.at[ridx_v.at[j]], add=True)
        plsc.subcore_barrier()

        pltpu.sync_copy(acc.at[pl.ds(r0, RPT)], s2_v)
        m = jnp.where(cid == 0, 1.0, 0.0).astype(_f32)
        lane = lax.iota(jnp.int32, 16)
        zcol = jnp.zeros((16,), jnp.int32)
        ocol = jnp.ones((16,), jnp.int32)

        def pool_body(g, _):
            base = g * 16
            rowi = base + lane
            s20 = plsc.load_gather(s2_v, [rowi, zcol])
            s21 = plsc.load_gather(s2_v, [rowi, ocol])
            bv = b_v[pl.ds(base, 16)]
            v0 = c_v[pl.ds(base, 16)] * s20 + m * p0_v[pl.ds(base, 16)]
            v1 = c_v[pl.ds(base, 16)] * s21 + m * p1_v[pl.ds(base, 16)]
            plsc.addupdate_scatter(pool_v, [bv, zcol], v0)
            plsc.addupdate_scatter(pool_v, [bv, ocol], v1)
            return _

        lax.fori_loop(0, RPT // 16, pool_body, None)
        pltpu.sync_copy(pool_v, pool_s.at[iota_v], add=True)
        plsc.subcore_barrier()

        @pl.when(sid == 0)
        def _():
            pltpu.sync_copy(pool_s, out_hbm.at[cid])

    return prop_pool


def _deg_spec():
    return pl.BlockSpec((NC, RB, DW), lambda i: (0, i, 0))


def kernel(x, senders, receivers, batch, num_graphs, W0, b0, W1, b1, W2, b2):
    send3 = senders.reshape(NW, NCH, CH)
    recv3 = receivers.reshape(NW, NCH, CH)
    b0_2d = jnp.broadcast_to(b0[None, :], (8, H0))
    b1_2d = jnp.broadcast_to(b1[None, :], (8, H1))
    b2_16 = jnp.zeros((8, 16), _f32).at[:, :D_OUT].set(b2[None, :])
    w2_16 = jnp.zeros((H1, 16), _f32).at[:, :D_OUT].set(W2)
    batch_pad = jnp.concatenate(
        [batch, jnp.full((N_P - N,), 127, batch.dtype)])
    iota128 = jnp.arange(128, dtype=jnp.int32)

    ones_s = jnp.zeros((CH, DW), _f32).at[:, :4].set(1.0)
    ones_r = jnp.zeros((CH, DW), _f32).at[:, 4:].set(1.0)
    zeros8 = jnp.zeros((ZB, DW), _f32)
    zeros16 = jnp.zeros((ZB, 16), _f32)
    zeros64 = jnp.zeros((ZB, H0), _f32)
    zeros80 = jnp.zeros((ZB, TW), _f32)

    deg = _make_deg_kernel()(send3, recv3, ones_s, ones_r, zeros8)

    h0raw = pl.pallas_call(
        _tc1a_body,
        grid=(NRB,),
        in_specs=[
            pl.BlockSpec((RB, D_IN), lambda i: (i, 0)),
            pl.BlockSpec((D_IN, H0), lambda i: (0, 0)),
            pl.BlockSpec((8, H0), lambda i: (0, 0)),
        ],
        out_specs=pl.BlockSpec((RB, H0), lambda i: (i, 0)),
        out_shape=jax.ShapeDtypeStruct((N, H0), _f32),
    )(x, W0, b0_2d)

    t0 = pl.pallas_call(
        _tc1b_body,
        grid=(NRB,),
        in_specs=[
            pl.BlockSpec((RB, H0), lambda i: (i, 0)),
            _deg_spec(),
        ],
        out_specs=pl.BlockSpec((RB, TW), lambda i: (i, 0)),
        out_shape=jax.ShapeDtypeStruct((N_P, TW), _f32),
    )(h0raw, deg)

    s0p = _make_prop_kernel(TW)(t0, send3, recv3, zeros80)

    t1, aux = pl.pallas_call(
        _tc2_body,
        out_shape=[
            jax.ShapeDtypeStruct((N_P, H0), _f32),
            jax.ShapeDtypeStruct((8, N_P), _f32),
        ],
    )(s0p, deg, b2_16)

    s1p = _make_prop_kernel(H0)(t1, send3, recv3, zeros64)

    t2 = pl.pallas_call(
        _tc3_body,
        out_shape=jax.ShapeDtypeStruct((N_P, 16), _f32),
    )(s1p, aux, W1, b1_2d, w2_16)

    pooled = _make_prop_pool_kernel()(
        t2, send3, recv3, zeros16, aux, batch_pad, iota128)

    return (pooled[0] + pooled[1])[:G, :D_OUT]


# speedup vs baseline: 17.0978x; 1.0049x over previous
import functools

import jax
import jax.numpy as jnp
from jax import lax
from jax.experimental import pallas as pl
from jax.experimental.pallas import tpu as pltpu
from jax.experimental.pallas import tpu_sc as plsc

N = 10000
E = 320000
D_IN = 128
H0 = 64
H1 = 128
D_OUT = 2
G = 100

NC = 2
NS = 16
NW = NC * NS
EPT = E // NW
CH = 125
NCH = EPT // CH
NB = 4
NGRP = NCH // NB
N_P = 10240
RPT = N_P // NS
ZB = 128

TW = 72
DW = 8

RB = 2000
NRB = N // RB

_f32 = jnp.float32


def _sc_mesh():
    return plsc.VectorSubcoreMesh(
        core_axis_name="c", subcore_axis_name="s", num_cores=NC, num_subcores=NS
    )


def _make_deg_kernel():
    @functools.partial(
        pl.kernel,
        out_type=jax.ShapeDtypeStruct((NC, N_P, DW), _f32),
        mesh=_sc_mesh(),
        compiler_params=pltpu.CompilerParams(use_tc_tiling_on_sc=False),
        scratch_types=[
            pltpu.VMEM((NCH, CH), jnp.int32),
            pltpu.VMEM((NCH, CH), jnp.int32),
            pltpu.VMEM((CH, DW), _f32),
            pltpu.VMEM((CH, DW), _f32),
            pltpu.VMEM((ZB, DW), _f32),
            pltpu.VMEM_SHARED((N_P, DW), _f32),
        ],
    )
    def deg_kernel(send_hbm, recv_hbm, ones_s_hbm, ones_r_hbm, zeros_hbm,
                   deg_hbm, sidx_v, ridx_v, ones_s_v, ones_r_v, buf_v, acc):
        cid = lax.axis_index("c")
        sid = lax.axis_index("s")
        wid = cid * NS + sid
        r0 = sid * RPT
        pltpu.sync_copy(zeros_hbm, buf_v)
        for z in range(RPT // ZB):
            pltpu.sync_copy(buf_v, acc.at[pl.ds(r0 + z * ZB, ZB)])
        pltpu.sync_copy(ones_s_hbm, ones_s_v)
        pltpu.sync_copy(ones_r_hbm, ones_r_v)
        pltpu.sync_copy(send_hbm.at[wid], sidx_v)
        pltpu.sync_copy(recv_hbm.at[wid], ridx_v)
        plsc.subcore_barrier()

        def body(j, _):
            pltpu.sync_copy(ones_s_v, acc.at[sidx_v.at[j]], add=True)
            pltpu.sync_copy(ones_r_v, acc.at[ridx_v.at[j]], add=True)
            return _

        lax.fori_loop(0, NCH, body, None)
        plsc.subcore_barrier()
        for z in range(RPT // ZB):
            pltpu.sync_copy(acc.at[pl.ds(r0 + z * ZB, ZB)], buf_v)
            pltpu.sync_copy(buf_v, deg_hbm.at[cid, pl.ds(r0 + z * ZB, ZB)])

    return deg_kernel


def _make_prop_kernel(D):
    @functools.partial(
        pl.kernel,
        out_type=jax.ShapeDtypeStruct((NC, N_P, D), _f32),
        mesh=_sc_mesh(),
        compiler_params=pltpu.CompilerParams(use_tc_tiling_on_sc=False),
        scratch_types=[
            pltpu.VMEM((NCH, CH), jnp.int32),
            pltpu.VMEM((NCH, CH), jnp.int32),
            [pltpu.VMEM((CH, D), _f32) for _ in range(NB)],
            pltpu.VMEM((ZB, D), _f32),
            pltpu.VMEM_SHARED((N_P, D), _f32),
            [pltpu.SemaphoreType.DMA for _ in range(NB)],
        ],
    )
    def prop_kernel(table_hbm, send_hbm, recv_hbm, zeros_hbm, out_hbm,
                    sidx_v, ridx_v, rows, buf_v, acc, sems):
        cid = lax.axis_index("c")
        sid = lax.axis_index("s")
        wid = cid * NS + sid
        r0 = sid * RPT
        pltpu.sync_copy(zeros_hbm, buf_v)
        for z in range(RPT // ZB):
            pltpu.sync_copy(buf_v, acc.at[pl.ds(r0 + z * ZB, ZB)])
        pltpu.sync_copy(send_hbm.at[wid], sidx_v)
        pltpu.sync_copy(recv_hbm.at[wid], ridx_v)
        plsc.subcore_barrier()

        for b in range(NB):
            pltpu.async_copy(table_hbm.at[sidx_v.at[b]], rows[b], sems[b])

        def grp(g, _):
            for b in range(NB):
                j = g * NB + b
                pltpu.make_async_copy(
                    table_hbm.at[sidx_v.at[j]], rows[b], sems[b]).wait()
                pltpu.sync_copy(rows[b], acc.at[ridx_v.at[j]], add=True)
                pltpu.async_copy(
                    table_hbm.at[sidx_v.at[j + NB]], rows[b], sems[b])
            return _

        lax.fori_loop(0, NGRP - 1, grp, None)
        for b in range(NB):
            j = (NGRP - 1) * NB + b
            pltpu.make_async_copy(
                table_hbm.at[sidx_v.at[j]], rows[b], sems[b]).wait()
            pltpu.sync_copy(rows[b], acc.at[ridx_v.at[j]], add=True)
        plsc.subcore_barrier()
        for z in range(RPT // ZB):
            pltpu.sync_copy(acc.at[pl.ds(r0 + z * ZB, ZB)], buf_v)
            pltpu.sync_copy(buf_v, out_hbm.at[cid, pl.ds(r0 + z * ZB, ZB)])

    return prop_kernel


def _scales(d_ref):
    out_deg = d_ref[0, :, 0] + d_ref[1, :, 0]
    in_deg = d_ref[0, :, 4] + d_ref[1, :, 4]
    a = lax.rsqrt(jnp.maximum(out_deg, 1.0))
    c = lax.rsqrt(jnp.maximum(in_deg, 1.0))
    return a, c


def _tc1a_body(x_ref, w0_ref, b0_ref, h_ref):
    h = jnp.dot(x_ref[...], w0_ref[...], preferred_element_type=_f32)
    h_ref[...] = h + b0_ref[0][None, :]


def _tc1b_body(h_ref, d_ref, t0_ref):
    a, _ = _scales(d_ref)
    h = h_ref[...]
    t0_ref[...] = jnp.concatenate(
        [h * a[:, None], a[:, None], jnp.zeros((RB, TW - H0 - 1), _f32)], axis=1
    )


def _tc2_body(s0_ref, d_ref, b2_ref, t1_ref, aux_ref):
    a, c = _scales(d_ref)
    s0 = s0_ref[0] + s0_ref[1]
    h0 = c[:, None] * s0[:, :H0]
    t1_ref[...] = a[:, None] * jnp.maximum(h0, 0.0)
    t = s0_ref[0, :, H0] + s0_ref[1, :, H0]
    ct = c * t
    b2m = b2_ref[...]
    aux_ref[...] = jnp.concatenate(
        [a[:, None], c[:, None], ct[:, None],
         (ct * b2m[0, 0])[:, None], (ct * b2m[0, 1])[:, None],
         jnp.zeros((RB, 3), _f32)], axis=1)


def _tc3_body(s1_ref, aux_ref, w1_ref, b1_ref, w2_ref, t2_ref):
    s1 = s1_ref[0] + s1_ref[1]
    aux = aux_ref[...]
    a, c, ct = aux[:, 0], aux[:, 1], aux[:, 2]
    h1 = jnp.dot(c[:, None] * s1, w1_ref[...], preferred_element_type=_f32)
    h1 = h1 + ct[:, None] * b1_ref[0][None, :]
    r1 = jnp.maximum(h1, 0.0)
    t2_ref[...] = jnp.dot(a[:, None] * r1, w2_ref[...], preferred_element_type=_f32)


def _make_prop_pool_kernel():
    D = 16

    @functools.partial(
        pl.kernel,
        out_type=jax.ShapeDtypeStruct((NC, 128, 16), _f32),
        mesh=_sc_mesh(),
        compiler_params=pltpu.CompilerParams(
            use_tc_tiling_on_sc=False, needs_layout_passes=False),
        scratch_types=[
            pltpu.VMEM((NCH, CH), jnp.int32),
            pltpu.VMEM((NCH, CH), jnp.int32),
            [pltpu.VMEM((CH, D), _f32) for _ in range(NB)],
            pltpu.VMEM((ZB, D), _f32),
            pltpu.VMEM((RPT, D), _f32),
            pltpu.VMEM((RPT, 8), _f32),
            pltpu.VMEM((RPT,), jnp.int32),
            pltpu.VMEM((128,), jnp.int32),
            pltpu.VMEM((128, 16), _f32),
            pltpu.VMEM_SHARED((N_P, D), _f32),
            pltpu.VMEM_SHARED((128, 16), _f32),
            [pltpu.SemaphoreType.DMA for _ in range(NB)],
        ],
    )
    def prop_pool(table_hbm, send_hbm, recv_hbm, zeros_hbm,
                  aux_hbm, batch_hbm, iota_hbm, out_hbm,
                  sidx_v, ridx_v, rows, buf_v, s2_v, aux_v, b_v,
                  iota_v, pool_v, acc, pool_s, sems):
        cid = lax.axis_index("c")
        sid = lax.axis_index("s")
        wid = cid * NS + sid
        r0 = sid * RPT
        pltpu.sync_copy(zeros_hbm, buf_v)
        for z in range(RPT // ZB):
            pltpu.sync_copy(buf_v, acc.at[pl.ds(r0 + z * ZB, ZB)])
        pltpu.sync_copy(zeros_hbm, pool_v)

        @pl.when(sid == 0)
        def _():
            pltpu.sync_copy(pool_v, pool_s)

        pltpu.sync_copy(aux_hbm.at[pl.ds(r0, RPT)], aux_v)
        pltpu.sync_copy(batch_hbm.at[pl.ds(r0, RPT)], b_v)
        pltpu.sync_copy(iota_hbm, iota_v)
        pltpu.sync_copy(send_hbm.at[wid], sidx_v)
        pltpu.sync_copy(recv_hbm.at[wid], ridx_v)
        plsc.subcore_barrier()

        for b in range(NB):
            pltpu.async_copy(table_hbm.at[sidx_v.at[b]], rows[b], sems[b])

        def grp(g, _):
            for b in range(NB):
                j = g * NB + b
                pltpu.make_async_copy(
                    table_hbm.at[sidx_v.at[j]], rows[b], sems[b]).wait()
                pltpu.sync_copy(rows[b], acc.at[ridx_v.at[j]], add=True)
                pltpu.async_copy(
                    table_hbm.at[sidx_v.at[j + NB]], rows[b], sems[b])
            return _

        lax.fori_loop(0, NGRP - 1, grp, None)
        for b in range(NB):
            j = (NGRP - 1) * NB + b
            pltpu.make_async_copy(
                table_hbm.at[sidx_v.at[j]], rows[b], sems[b]).wait()
            pltpu.sync_copy(rows[b], acc.at[ridx_v.at[j]], add=True)
        plsc.subcore_barrier()

        pltpu.sync_copy(acc.at[pl.ds(r0, RPT)], s2_v)
        m = jnp.where(cid == 0, 1.0, 0.0).astype(_f32)
        lane = lax.iota(jnp.int32, 16)
        zcol = jnp.zeros((16,), jnp.int32)
        ocol = jnp.ones((16,), jnp.int32)

        def pool_body(g, _):
            base = g * 16
            rowi = base + lane
            s20 = plsc.load_gather(s2_v, [rowi, zcol])
            s21 = plsc.load_gather(s2_v, [rowi, ocol])
            cv = plsc.load_gather(aux_v, [rowi, ocol])
            p0 = plsc.load_gather(aux_v, [rowi, zcol + 3])
            p1 = plsc.load_gather(aux_v, [rowi, zcol + 4])
            bv = b_v[pl.ds(base, 16)]
            v0 = cv * s20 + m * p0
            v1 = cv * s21 + m * p1
            plsc.addupdate_scatter(pool_v, [bv, zcol], v0)
            plsc.addupdate_scatter(pool_v, [bv, ocol], v1)
            return _

        lax.fori_loop(0, RPT // 16, pool_body, None)
        pltpu.sync_copy(pool_v, pool_s.at[iota_v], add=True)
        plsc.subcore_barrier()

        @pl.when(sid == 0)
        def _():
            pltpu.sync_copy(pool_s, out_hbm.at[cid])

    return prop_pool


def _deg_spec():
    return pl.BlockSpec((NC, RB, DW), lambda i: (0, i, 0))


def kernel(x, senders, receivers, batch, num_graphs, W0, b0, W1, b1, W2, b2):
    send3 = senders.reshape(NW, NCH, CH)
    recv3 = receivers.reshape(NW, NCH, CH)
    b0_2d = jnp.broadcast_to(b0[None, :], (8, H0))
    b1_2d = jnp.broadcast_to(b1[None, :], (8, H1))
    b2_16 = jnp.zeros((8, 16), _f32).at[:, :D_OUT].set(b2[None, :])
    w2_16 = jnp.zeros((H1, 16), _f32).at[:, :D_OUT].set(W2)
    batch_pad = jnp.concatenate(
        [batch, jnp.full((N_P - N,), 127, batch.dtype)])
    iota128 = jnp.arange(128, dtype=jnp.int32)

    ones_s = jnp.zeros((CH, DW), _f32).at[:, :4].set(1.0)
    ones_r = jnp.zeros((CH, DW), _f32).at[:, 4:].set(1.0)
    zeros8 = jnp.zeros((ZB, DW), _f32)
    zeros16 = jnp.zeros((ZB, 16), _f32)
    zeros64 = jnp.zeros((ZB, H0), _f32)
    zeros80 = jnp.zeros((ZB, TW), _f32)

    deg = _make_deg_kernel()(send3, recv3, ones_s, ones_r, zeros8)

    h0raw = pl.pallas_call(
        _tc1a_body,
        grid=(NRB,),
        in_specs=[
            pl.BlockSpec((RB, D_IN), lambda i: (i, 0)),
            pl.BlockSpec((D_IN, H0), lambda i: (0, 0)),
            pl.BlockSpec((8, H0), lambda i: (0, 0)),
        ],
        out_specs=pl.BlockSpec((RB, H0), lambda i: (i, 0)),
        out_shape=jax.ShapeDtypeStruct((N, H0), _f32),
    )(x, W0, b0_2d)

    t0 = pl.pallas_call(
        _tc1b_body,
        grid=(NRB,),
        in_specs=[
            pl.BlockSpec((RB, H0), lambda i: (i, 0)),
            _deg_spec(),
        ],
        out_specs=pl.BlockSpec((RB, TW), lambda i: (i, 0)),
        out_shape=jax.ShapeDtypeStruct((N_P, TW), _f32),
    )(h0raw, deg)

    s0p = _make_prop_kernel(TW)(t0, send3, recv3, zeros80)

    t1, aux = pl.pallas_call(
        _tc2_body,
        grid=(NRB,),
        in_specs=[
            pl.BlockSpec((NC, RB, TW), lambda i: (0, i, 0)),
            _deg_spec(),
            pl.BlockSpec((8, 16), lambda i: (0, 0)),
        ],
        out_specs=[
            pl.BlockSpec((RB, H0), lambda i: (i, 0)),
            pl.BlockSpec((RB, 8), lambda i: (i, 0)),
        ],
        out_shape=[
            jax.ShapeDtypeStruct((N_P, H0), _f32),
            jax.ShapeDtypeStruct((N_P, 8), _f32),
        ],
    )(s0p, deg, b2_16)

    s1p = _make_prop_kernel(H0)(t1, send3, recv3, zeros64)

    t2 = pl.pallas_call(
        _tc3_body,
        grid=(NRB,),
        in_specs=[
            pl.BlockSpec((NC, RB, H0), lambda i: (0, i, 0)),
            pl.BlockSpec((RB, 8), lambda i: (i, 0)),
            pl.BlockSpec((H0, H1), lambda i: (0, 0)),
            pl.BlockSpec((8, H1), lambda i: (0, 0)),
            pl.BlockSpec((H1, 16), lambda i: (0, 0)),
        ],
        out_specs=pl.BlockSpec((RB, 16), lambda i: (i, 0)),
        out_shape=jax.ShapeDtypeStruct((N_P, 16), _f32),
    )(s1p, aux, W1, b1_2d, w2_16)

    pooled = _make_prop_pool_kernel()(
        t2, send3, recv3, zeros16, aux, batch_pad, iota128)

    return (pooled[0] + pooled[1])[:G, :D_OUT]


# speedup vs baseline: 18.0342x; 1.0548x over previous
import functools

import jax
import jax.numpy as jnp
from jax import lax
from jax.experimental import pallas as pl
from jax.experimental.pallas import tpu as pltpu
from jax.experimental.pallas import tpu_sc as plsc

N = 10000
E = 320000
D_IN = 128
H0 = 64
H1 = 128
D_OUT = 2
G = 100

NC = 2
NS = 16
NW = NC * NS
EPT = E // NW
CH = 125
NCH = EPT // CH
NB = 4
NGRP = NCH // NB
N_P = 10240
RPT = N_P // NS
ZB = 128

TW = 72
DW = 8

RB = 2000
NRB = N // RB

_f32 = jnp.float32


def _sc_mesh():
    return plsc.VectorSubcoreMesh(
        core_axis_name="c", subcore_axis_name="s", num_cores=NC, num_subcores=NS
    )


def _make_deg_kernel():
    @functools.partial(
        pl.kernel,
        out_type=jax.ShapeDtypeStruct((NC, 2, N_P), _f32),
        mesh=_sc_mesh(),
        compiler_params=pltpu.CompilerParams(
            use_tc_tiling_on_sc=False, needs_layout_passes=False),
        scratch_types=[
            pltpu.VMEM((EPT,), jnp.int32),
            pltpu.VMEM((EPT,), jnp.int32),
            pltpu.VMEM((N_P,), _f32),
            pltpu.VMEM((N_P,), _f32),
            pltpu.VMEM((RPT,), _f32),
            pltpu.VMEM((RPT,), _f32),
            pltpu.VMEM_SHARED((NS, N_P), _f32),
        ],
    )
    def deg_kernel(send_hbm, recv_hbm, zerosn_hbm, deg_hbm,
                   sidx_v, ridx_v, degs_t, degr_t, tmp_v, acc_v, stage):
        cid = lax.axis_index("c")
        sid = lax.axis_index("s")
        wid = cid * NS + sid
        r0 = sid * RPT
        pltpu.sync_copy(send_hbm.at[wid], sidx_v)
        pltpu.sync_copy(recv_hbm.at[wid], ridx_v)
        pltpu.sync_copy(zerosn_hbm, degs_t)
        pltpu.sync_copy(zerosn_hbm, degr_t)
        ones16 = jnp.ones((16,), _f32)

        def count(k, _):
            base = k * 16
            plsc.addupdate_scatter(degs_t, [sidx_v[pl.ds(base, 16)]], ones16)
            plsc.addupdate_scatter(degr_t, [ridx_v[pl.ds(base, 16)]], ones16)
            return _

        lax.fori_loop(0, EPT // 16, count, None)

        for rnd, (tab, row) in enumerate(((degs_t, 0), (degr_t, 1))):
            if rnd:
                plsc.subcore_barrier()
            pltpu.sync_copy(tab, stage.at[sid])
            plsc.subcore_barrier()
            pltpu.sync_copy(stage.at[0, pl.ds(r0, RPT)], acc_v)
            for k in range(1, NS):
                pltpu.sync_copy(stage.at[k, pl.ds(r0, RPT)], tmp_v)

                def addup(q, _):
                    s = pl.ds(q * 16, 16)
                    acc_v[s] = acc_v[s] + tmp_v[s]
                    return _

                lax.fori_loop(0, RPT // 16, addup, None)
            pltpu.sync_copy(acc_v, deg_hbm.at[cid, row, pl.ds(r0, RPT)])

    return deg_kernel


def _make_prop_kernel(D):
    @functools.partial(
        pl.kernel,
        out_type=jax.ShapeDtypeStruct((NC, N_P, D), _f32),
        mesh=_sc_mesh(),
        compiler_params=pltpu.CompilerParams(use_tc_tiling_on_sc=False),
        scratch_types=[
            pltpu.VMEM((NCH, CH), jnp.int32),
            pltpu.VMEM((NCH, CH), jnp.int32),
            [pltpu.VMEM((CH, D), _f32) for _ in range(NB)],
            pltpu.VMEM((ZB, D), _f32),
            pltpu.VMEM_SHARED((N_P, D), _f32),
            [pltpu.SemaphoreType.DMA for _ in range(NB)],
        ],
    )
    def prop_kernel(table_hbm, send_hbm, recv_hbm, zeros_hbm, out_hbm,
                    sidx_v, ridx_v, rows, buf_v, acc, sems):
        cid = lax.axis_index("c")
        sid = lax.axis_index("s")
        wid = cid * NS + sid
        r0 = sid * RPT
        pltpu.sync_copy(zeros_hbm, buf_v)
        for z in range(RPT // ZB):
            pltpu.sync_copy(buf_v, acc.at[pl.ds(r0 + z * ZB, ZB)])
        pltpu.sync_copy(send_hbm.at[wid], sidx_v)
        pltpu.sync_copy(recv_hbm.at[wid], ridx_v)
        plsc.subcore_barrier()

        for b in range(NB):
            pltpu.async_copy(table_hbm.at[sidx_v.at[b]], rows[b], sems[b])

        def grp(g, _):
            for b in range(NB):
                j = g * NB + b
                pltpu.make_async_copy(
                    table_hbm.at[sidx_v.at[j]], rows[b], sems[b]).wait()
                pltpu.sync_copy(rows[b], acc.at[ridx_v.at[j]], add=True)
                pltpu.async_copy(
                    table_hbm.at[sidx_v.at[j + NB]], rows[b], sems[b])
            return _

        lax.fori_loop(0, NGRP - 1, grp, None)
        for b in range(NB):
            j = (NGRP - 1) * NB + b
            pltpu.make_async_copy(
                table_hbm.at[sidx_v.at[j]], rows[b], sems[b]).wait()
            pltpu.sync_copy(rows[b], acc.at[ridx_v.at[j]], add=True)
        plsc.subcore_barrier()
        for z in range(RPT // ZB):
            pltpu.sync_copy(acc.at[pl.ds(r0 + z * ZB, ZB)], buf_v)
            pltpu.sync_copy(buf_v, out_hbm.at[cid, pl.ds(r0 + z * ZB, ZB)])

    return prop_kernel


def _scales(d_ref):
    out_deg = d_ref[0, 0] + d_ref[1, 0]
    in_deg = d_ref[0, 1] + d_ref[1, 1]
    a = lax.rsqrt(jnp.maximum(out_deg, 1.0))
    c = lax.rsqrt(jnp.maximum(in_deg, 1.0))
    return a, c


def _tc1a_body(x_ref, w0_ref, b0_ref, h_ref):
    h = jnp.dot(x_ref[...], w0_ref[...], preferred_element_type=_f32)
    h_ref[...] = h + b0_ref[0][None, :]


def _tc1b_body(h_ref, d_ref, t0_ref):
    a, _ = _scales(d_ref)
    h = jnp.concatenate([h_ref[...], jnp.zeros((N_P - N, H0), _f32)], axis=0)
    t0_ref[...] = jnp.concatenate(
        [h * a[:, None], a[:, None], jnp.zeros((N_P, TW - H0 - 1), _f32)],
        axis=1)


def _tc2_body(s0_ref, d_ref, b2_ref, t1_ref, aux_ref):
    a, c = _scales(d_ref)
    s0 = s0_ref[0] + s0_ref[1]
    h0 = c[:, None] * s0[:, :H0]
    t1_ref[...] = a[:, None] * jnp.maximum(h0, 0.0)
    t = s0_ref[0, :, H0] + s0_ref[1, :, H0]
    ct = c * t
    b2m = b2_ref[...]
    aux_ref[...] = jnp.concatenate(
        [a[:, None], c[:, None], ct[:, None],
         (ct * b2m[0, 0])[:, None], (ct * b2m[0, 1])[:, None],
         jnp.zeros((N_P, 3), _f32)], axis=1)


def _tc3_body(s1_ref, aux_ref, w1_ref, b1_ref, w2_ref, t2_ref):
    s1 = s1_ref[0] + s1_ref[1]
    aux = aux_ref[...]
    a, c, ct = aux[:, 0], aux[:, 1], aux[:, 2]
    h1 = jnp.dot(c[:, None] * s1, w1_ref[...], preferred_element_type=_f32)
    h1 = h1 + ct[:, None] * b1_ref[0][None, :]
    r1 = jnp.maximum(h1, 0.0)
    t2_ref[...] = jnp.dot(a[:, None] * r1, w2_ref[...], preferred_element_type=_f32)


def _make_prop_pool_kernel():
    D = 16

    @functools.partial(
        pl.kernel,
        out_type=jax.ShapeDtypeStruct((NC, 128, 16), _f32),
        mesh=_sc_mesh(),
        compiler_params=pltpu.CompilerParams(
            use_tc_tiling_on_sc=False, needs_layout_passes=False),
        scratch_types=[
            pltpu.VMEM((NCH, CH), jnp.int32),
            pltpu.VMEM((NCH, CH), jnp.int32),
            [pltpu.VMEM((CH, D), _f32) for _ in range(NB)],
            pltpu.VMEM((ZB, D), _f32),
            pltpu.VMEM((RPT, D), _f32),
            pltpu.VMEM((RPT, 8), _f32),
            pltpu.VMEM((RPT,), jnp.int32),
            pltpu.VMEM((128,), jnp.int32),
            pltpu.VMEM((128, 16), _f32),
            pltpu.VMEM_SHARED((N_P, D), _f32),
            pltpu.VMEM_SHARED((128, 16), _f32),
            [pltpu.SemaphoreType.DMA for _ in range(NB)],
        ],
    )
    def prop_pool(table_hbm, send_hbm, recv_hbm, zeros_hbm,
                  aux_hbm, batch_hbm, iota_hbm, out_hbm,
                  sidx_v, ridx_v, rows, buf_v, s2_v, aux_v, b_v,
                  iota_v, pool_v, acc, pool_s, sems):
        cid = lax.axis_index("c")
        sid = lax.axis_index("s")
        wid = cid * NS + sid
        r0 = sid * RPT
        pltpu.sync_copy(zeros_hbm, buf_v)
        for z in range(RPT // ZB):
            pltpu.sync_copy(buf_v, acc.at[pl.ds(r0 + z * ZB, ZB)])
        pltpu.sync_copy(zeros_hbm, pool_v)

        @pl.when(sid == 0)
        def _():
            pltpu.sync_copy(pool_v, pool_s)

        pltpu.sync_copy(aux_hbm.at[pl.ds(r0, RPT)], aux_v)
        pltpu.sync_copy(batch_hbm.at[pl.ds(r0, RPT)], b_v)
        pltpu.sync_copy(iota_hbm, iota_v)
        pltpu.sync_copy(send_hbm.at[wid], sidx_v)
        pltpu.sync_copy(recv_hbm.at[wid], ridx_v)
        plsc.subcore_barrier()

        for b in range(NB):
            pltpu.async_copy(table_hbm.at[sidx_v.at[b]], rows[b], sems[b])

        def grp(g, _):
            for b in range(NB):
                j = g * NB + b
                pltpu.make_async_copy(
                    table_hbm.at[sidx_v.at[j]], rows[b], sems[b]).wait()
                pltpu.sync_copy(rows[b], acc.at[ridx_v.at[j]], add=True)
                pltpu.async_copy(
                    table_hbm.at[sidx_v.at[j + NB]], rows[b], sems[b])
            return _

        lax.fori_loop(0, NGRP - 1, grp, None)
        for b in range(NB):
            j = (NGRP - 1) * NB + b
            pltpu.make_async_copy(
                table_hbm.at[sidx_v.at[j]], rows[b], sems[b]).wait()
            pltpu.sync_copy(rows[b], acc.at[ridx_v.at[j]], add=True)
        plsc.subcore_barrier()

        pltpu.sync_copy(acc.at[pl.ds(r0, RPT)], s2_v)
        m = jnp.where(cid == 0, 1.0, 0.0).astype(_f32)
        lane = lax.iota(jnp.int32, 16)
        zcol = jnp.zeros((16,), jnp.int32)
        ocol = jnp.ones((16,), jnp.int32)

        def pool_body(g, _):
            base = g * 16
            rowi = base + lane
            s20 = plsc.load_gather(s2_v, [rowi, zcol])
            s21 = plsc.load_gather(s2_v, [rowi, ocol])
            cv = plsc.load_gather(aux_v, [rowi, ocol])
            p0 = plsc.load_gather(aux_v, [rowi, zcol + 3])
            p1 = plsc.load_gather(aux_v, [rowi, zcol + 4])
            bv = b_v[pl.ds(base, 16)]
            v0 = cv * s20 + m * p0
            v1 = cv * s21 + m * p1
            plsc.addupdate_scatter(pool_v, [bv, zcol], v0)
            plsc.addupdate_scatter(pool_v, [bv, ocol], v1)
            return _

        lax.fori_loop(0, RPT // 16, pool_body, None)
        pltpu.sync_copy(pool_v, pool_s.at[iota_v], add=True)
        plsc.subcore_barrier()

        @pl.when(sid == 0)
        def _():
            pltpu.sync_copy(pool_s, out_hbm.at[cid])

    return prop_pool


def _deg_spec():
    return pl.BlockSpec((NC, 2, N_P), lambda i: (0, 0, 0))


def kernel(x, senders, receivers, batch, num_graphs, W0, b0, W1, b1, W2, b2):
    send3 = senders.reshape(NW, NCH, CH)
    recv3 = receivers.reshape(NW, NCH, CH)
    send2 = senders.reshape(NW, EPT)
    recv2 = receivers.reshape(NW, EPT)
    b0_2d = jnp.broadcast_to(b0[None, :], (8, H0))
    b1_2d = jnp.broadcast_to(b1[None, :], (8, H1))
    b2_16 = jnp.zeros((8, 16), _f32).at[:, :D_OUT].set(b2[None, :])
    w2_16 = jnp.zeros((H1, 16), _f32).at[:, :D_OUT].set(W2)
    batch_pad = jnp.concatenate(
        [batch, jnp.full((N_P - N,), 127, batch.dtype)])
    iota128 = jnp.arange(128, dtype=jnp.int32)

    zerosn = jnp.zeros((N_P,), _f32)
    zeros16 = jnp.zeros((ZB, 16), _f32)
    zeros64 = jnp.zeros((ZB, H0), _f32)
    zeros80 = jnp.zeros((ZB, TW), _f32)

    deg = _make_deg_kernel()(send2, recv2, zerosn)

    h0raw = pl.pallas_call(
        _tc1a_body,
        grid=(NRB,),
        in_specs=[
            pl.BlockSpec((RB, D_IN), lambda i: (i, 0)),
            pl.BlockSpec((D_IN, H0), lambda i: (0, 0)),
            pl.BlockSpec((8, H0), lambda i: (0, 0)),
        ],
        out_specs=pl.BlockSpec((RB, H0), lambda i: (i, 0)),
        out_shape=jax.ShapeDtypeStruct((N, H0), _f32),
    )(x, W0, b0_2d)

    t0 = pl.pallas_call(
        _tc1b_body,
        out_shape=jax.ShapeDtypeStruct((N_P, TW), _f32),
    )(h0raw, deg)

    s0p = _make_prop_kernel(TW)(t0, send3, recv3, zeros80)

    t1, aux = pl.pallas_call(
        _tc2_body,
        out_shape=[
            jax.ShapeDtypeStruct((N_P, H0), _f32),
            jax.ShapeDtypeStruct((N_P, 8), _f32),
        ],
    )(s0p, deg, b2_16)

    s1p = _make_prop_kernel(H0)(t1, send3, recv3, zeros64)

    t2 = pl.pallas_call(
        _tc3_body,
        grid=(NRB,),
        in_specs=[
            pl.BlockSpec((NC, RB, H0), lambda i: (0, i, 0)),
            pl.BlockSpec((RB, 8), lambda i: (i, 0)),
            pl.BlockSpec((H0, H1), lambda i: (0, 0)),
            pl.BlockSpec((8, H1), lambda i: (0, 0)),
            pl.BlockSpec((H1, 16), lambda i: (0, 0)),
        ],
        out_specs=pl.BlockSpec((RB, 16), lambda i: (i, 0)),
        out_shape=jax.ShapeDtypeStruct((N_P, 16), _f32),
    )(s1p, aux, W1, b1_2d, w2_16)

    pooled = _make_prop_pool_kernel()(
        t2, send3, recv3, zeros16, aux, batch_pad, iota128)

    return (pooled[0] + pooled[1])[:G, :D_OUT]


# speedup vs baseline: 18.1698x; 1.0075x over previous
import functools

import jax
import jax.numpy as jnp
from jax import lax
from jax.experimental import pallas as pl
from jax.experimental.pallas import tpu as pltpu
from jax.experimental.pallas import tpu_sc as plsc

N = 10000
E = 320000
D_IN = 128
H0 = 64
H1 = 128
D_OUT = 2
G = 100

NC = 2
NS = 16
NW = NC * NS
EPT = E // NW
CH = 125
NCH = EPT // CH
NB = 4
NGRP = NCH // NB
N_P = 10240
RPT = N_P // NS
ZB = 128

TW = 72
DW = 8

RB = 2000
NRB = N // RB

_f32 = jnp.float32


def _sc_mesh():
    return plsc.VectorSubcoreMesh(
        core_axis_name="c", subcore_axis_name="s", num_cores=NC, num_subcores=NS
    )


def _make_deg_kernel():
    @functools.partial(
        pl.kernel,
        out_type=jax.ShapeDtypeStruct((NC, 2, N_P), _f32),
        mesh=_sc_mesh(),
        compiler_params=pltpu.CompilerParams(
            use_tc_tiling_on_sc=False, needs_layout_passes=False),
        scratch_types=[
            pltpu.VMEM((EPT,), jnp.int32),
            pltpu.VMEM((EPT,), jnp.int32),
            pltpu.VMEM((N_P,), _f32),
            pltpu.VMEM((N_P,), _f32),
            pltpu.VMEM((RPT,), _f32),
            pltpu.VMEM((RPT,), _f32),
            pltpu.VMEM_SHARED((NS, N_P), _f32),
        ],
    )
    def deg_kernel(send_hbm, recv_hbm, zerosn_hbm, deg_hbm,
                   sidx_v, ridx_v, degs_t, degr_t, tmp_v, acc_v, stage):
        cid = lax.axis_index("c")
        sid = lax.axis_index("s")
        wid = cid * NS + sid
        r0 = sid * RPT
        pltpu.sync_copy(send_hbm.at[wid], sidx_v)
        pltpu.sync_copy(recv_hbm.at[wid], ridx_v)
        pltpu.sync_copy(zerosn_hbm, degs_t)
        pltpu.sync_copy(zerosn_hbm, degr_t)
        ones16 = jnp.ones((16,), _f32)

        def count(k, _):
            base = k * 16
            plsc.addupdate_scatter(degs_t, [sidx_v[pl.ds(base, 16)]], ones16)
            plsc.addupdate_scatter(degr_t, [ridx_v[pl.ds(base, 16)]], ones16)
            return _

        lax.fori_loop(0, EPT // 16, count, None)

        for rnd, (tab, row) in enumerate(((degs_t, 0), (degr_t, 1))):
            if rnd:
                plsc.subcore_barrier()
            pltpu.sync_copy(tab, stage.at[sid])
            plsc.subcore_barrier()
            pltpu.sync_copy(stage.at[0, pl.ds(r0, RPT)], acc_v)
            for k in range(1, NS):
                pltpu.sync_copy(stage.at[k, pl.ds(r0, RPT)], tmp_v)

                def addup(q, _):
                    s = pl.ds(q * 16, 16)
                    acc_v[s] = acc_v[s] + tmp_v[s]
                    return _

                lax.fori_loop(0, RPT // 16, addup, None)
            pltpu.sync_copy(acc_v, deg_hbm.at[cid, row, pl.ds(r0, RPT)])

    return deg_kernel


def _make_prop_kernel(D):
    @functools.partial(
        pl.kernel,
        out_type=jax.ShapeDtypeStruct((NC, N_P, D), _f32),
        mesh=_sc_mesh(),
        compiler_params=pltpu.CompilerParams(use_tc_tiling_on_sc=False),
        scratch_types=[
            pltpu.VMEM((NCH, CH), jnp.int32),
            pltpu.VMEM((NCH, CH), jnp.int32),
            [pltpu.VMEM((CH, D), _f32) for _ in range(NB)],
            pltpu.VMEM((ZB, D), _f32),
            pltpu.VMEM_SHARED((N_P, D), _f32),
            [pltpu.SemaphoreType.DMA for _ in range(NB)],
        ],
    )
    def prop_kernel(table_hbm, send_hbm, recv_hbm, zeros_hbm, out_hbm,
                    sidx_v, ridx_v, rows, buf_v, acc, sems):
        cid = lax.axis_index("c")
        sid = lax.axis_index("s")
        wid = cid * NS + sid
        r0 = sid * RPT
        pltpu.sync_copy(zeros_hbm, buf_v)
        for z in range(RPT // ZB):
            pltpu.sync_copy(buf_v, acc.at[pl.ds(r0 + z * ZB, ZB)])
        pltpu.sync_copy(send_hbm.at[wid], sidx_v)
        pltpu.sync_copy(recv_hbm.at[wid], ridx_v)
        plsc.subcore_barrier()

        for b in range(NB):
            pltpu.async_copy(table_hbm.at[sidx_v.at[b]], rows[b], sems[b])

        def grp(g, _):
            for b in range(NB):
                j = g * NB + b
                pltpu.make_async_copy(
                    table_hbm.at[sidx_v.at[j]], rows[b], sems[b]).wait()
                pltpu.sync_copy(rows[b], acc.at[ridx_v.at[j]], add=True)
                pltpu.async_copy(
                    table_hbm.at[sidx_v.at[j + NB]], rows[b], sems[b])
            return _

        lax.fori_loop(0, NGRP - 1, grp, None)
        for b in range(NB):
            j = (NGRP - 1) * NB + b
            pltpu.make_async_copy(
                table_hbm.at[sidx_v.at[j]], rows[b], sems[b]).wait()
            pltpu.sync_copy(rows[b], acc.at[ridx_v.at[j]], add=True)
        plsc.subcore_barrier()
        pltpu.sync_copy(acc.at[pl.ds(r0, RPT)], out_hbm.at[cid, pl.ds(r0, RPT)])

    return prop_kernel


def _scales(d_ref):
    out_deg = d_ref[0, 0] + d_ref[1, 0]
    in_deg = d_ref[0, 1] + d_ref[1, 1]
    a = lax.rsqrt(jnp.maximum(out_deg, 1.0))
    c = lax.rsqrt(jnp.maximum(in_deg, 1.0))
    return a, c


def _tc1_body(x_ref, w0_ref, b0_ref, d_ref, t0_ref):
    a, _ = _scales(d_ref)
    h = jnp.dot(x_ref[...], w0_ref[...], preferred_element_type=_f32)
    h = h + b0_ref[0][None, :]
    h = jnp.concatenate([h, jnp.zeros((N_P - N, H0), _f32)], axis=0)
    t0_ref[...] = jnp.concatenate(
        [h * a[:, None], a[:, None], jnp.zeros((N_P, TW - H0 - 1), _f32)],
        axis=1)


def _tc2_body(s0_ref, d_ref, b2_ref, t1_ref, aux_ref):
    a, c = _scales(d_ref)
    s0 = s0_ref[0] + s0_ref[1]
    h0 = c[:, None] * s0[:, :H0]
    t1_ref[...] = a[:, None] * jnp.maximum(h0, 0.0)
    t = s0_ref[0, :, H0] + s0_ref[1, :, H0]
    ct = c * t
    b2m = b2_ref[...]
    aux_ref[...] = jnp.concatenate(
        [a[:, None], c[:, None], ct[:, None],
         (ct * b2m[0, 0])[:, None], (ct * b2m[0, 1])[:, None],
         jnp.zeros((N_P, 3), _f32)], axis=1)


def _tc3_body(s1_ref, aux_ref, w1_ref, b1_ref, w2_ref, t2_ref):
    s1 = s1_ref[0] + s1_ref[1]
    aux = aux_ref[...]
    a, c, ct = aux[:, 0], aux[:, 1], aux[:, 2]
    h1 = jnp.dot(c[:, None] * s1, w1_ref[...], preferred_element_type=_f32)
    h1 = h1 + ct[:, None] * b1_ref[0][None, :]
    r1 = jnp.maximum(h1, 0.0)
    t2_ref[...] = jnp.dot(a[:, None] * r1, w2_ref[...], preferred_element_type=_f32)


def _make_prop_pool_kernel():
    D = 16

    @functools.partial(
        pl.kernel,
        out_type=jax.ShapeDtypeStruct((NC, 128, 16), _f32),
        mesh=_sc_mesh(),
        compiler_params=pltpu.CompilerParams(
            use_tc_tiling_on_sc=False, needs_layout_passes=False),
        scratch_types=[
            pltpu.VMEM((NCH, CH), jnp.int32),
            pltpu.VMEM((NCH, CH), jnp.int32),
            [pltpu.VMEM((CH, D), _f32) for _ in range(NB)],
            pltpu.VMEM((ZB, D), _f32),
            pltpu.VMEM((RPT, D), _f32),
            pltpu.VMEM((RPT, 8), _f32),
            pltpu.VMEM((RPT,), jnp.int32),
            pltpu.VMEM((128,), jnp.int32),
            pltpu.VMEM((128, 16), _f32),
            pltpu.VMEM_SHARED((N_P, D), _f32),
            pltpu.VMEM_SHARED((128, 16), _f32),
            [pltpu.SemaphoreType.DMA for _ in range(NB)],
        ],
    )
    def prop_pool(table_hbm, send_hbm, recv_hbm, zeros_hbm,
                  aux_hbm, batch_hbm, iota_hbm, out_hbm,
                  sidx_v, ridx_v, rows, buf_v, s2_v, aux_v, b_v,
                  iota_v, pool_v, acc, pool_s, sems):
        cid = lax.axis_index("c")
        sid = lax.axis_index("s")
        wid = cid * NS + sid
        r0 = sid * RPT
        pltpu.sync_copy(zeros_hbm, buf_v)
        for z in range(RPT // ZB):
            pltpu.sync_copy(buf_v, acc.at[pl.ds(r0 + z * ZB, ZB)])
        pltpu.sync_copy(zeros_hbm, pool_v)

        @pl.when(sid == 0)
        def _():
            pltpu.sync_copy(pool_v, pool_s)

        pltpu.sync_copy(aux_hbm.at[pl.ds(r0, RPT)], aux_v)
        pltpu.sync_copy(batch_hbm.at[pl.ds(r0, RPT)], b_v)
        pltpu.sync_copy(iota_hbm, iota_v)
        pltpu.sync_copy(send_hbm.at[wid], sidx_v)
        pltpu.sync_copy(recv_hbm.at[wid], ridx_v)
        plsc.subcore_barrier()

        for b in range(NB):
            pltpu.async_copy(table_hbm.at[sidx_v.at[b]], rows[b], sems[b])

        def grp(g, _):
            for b in range(NB):
                j = g * NB + b
                pltpu.make_async_copy(
                    table_hbm.at[sidx_v.at[j]], rows[b], sems[b]).wait()
                pltpu.sync_copy(rows[b], acc.at[ridx_v.at[j]], add=True)
                pltpu.async_copy(
                    table_hbm.at[sidx_v.at[j + NB]], rows[b], sems[b])
            return _

        lax.fori_loop(0, NGRP - 1, grp, None)
        for b in range(NB):
            j = (NGRP - 1) * NB + b
            pltpu.make_async_copy(
                table_hbm.at[sidx_v.at[j]], rows[b], sems[b]).wait()
            pltpu.sync_copy(rows[b], acc.at[ridx_v.at[j]], add=True)
        plsc.subcore_barrier()

        pltpu.sync_copy(acc.at[pl.ds(r0, RPT)], s2_v)
        m = jnp.where(cid == 0, 1.0, 0.0).astype(_f32)
        lane = lax.iota(jnp.int32, 16)
        zcol = jnp.zeros((16,), jnp.int32)
        ocol = jnp.ones((16,), jnp.int32)

        def pool_body(g, _):
            base = g * 16
            rowi = base + lane
            s20 = plsc.load_gather(s2_v, [rowi, zcol])
            s21 = plsc.load_gather(s2_v, [rowi, ocol])
            cv = plsc.load_gather(aux_v, [rowi, ocol])
            p0 = plsc.load_gather(aux_v, [rowi, zcol + 3])
            p1 = plsc.load_gather(aux_v, [rowi, zcol + 4])
            bv = b_v[pl.ds(base, 16)]
            v0 = cv * s20 + m * p0
            v1 = cv * s21 + m * p1
            plsc.addupdate_scatter(pool_v, [bv, zcol], v0)
            plsc.addupdate_scatter(pool_v, [bv, ocol], v1)
            return _

        lax.fori_loop(0, RPT // 16, pool_body, None)
        pltpu.sync_copy(pool_v, pool_s.at[iota_v], add=True)
        plsc.subcore_barrier()

        @pl.when(sid == 0)
        def _():
            pltpu.sync_copy(pool_s, out_hbm.at[cid])

    return prop_pool


def _deg_spec():
    return pl.BlockSpec((NC, 2, N_P), lambda i: (0, 0, 0))


def kernel(x, senders, receivers, batch, num_graphs, W0, b0, W1, b1, W2, b2):
    send3 = senders.reshape(NW, NCH, CH)
    recv3 = receivers.reshape(NW, NCH, CH)
    send2 = senders.reshape(NW, EPT)
    recv2 = receivers.reshape(NW, EPT)
    b0_2d = jnp.broadcast_to(b0[None, :], (8, H0))
    b1_2d = jnp.broadcast_to(b1[None, :], (8, H1))
    b2_16 = jnp.zeros((8, 16), _f32).at[:, :D_OUT].set(b2[None, :])
    w2_16 = jnp.zeros((H1, 16), _f32).at[:, :D_OUT].set(W2)
    batch_pad = jnp.concatenate(
        [batch, jnp.full((N_P - N,), 127, batch.dtype)])
    iota128 = jnp.arange(128, dtype=jnp.int32)

    zerosn = jnp.zeros((N_P,), _f32)
    zeros16 = jnp.zeros((ZB, 16), _f32)
    zeros64 = jnp.zeros((ZB, H0), _f32)
    zeros80 = jnp.zeros((ZB, TW), _f32)

    deg = _make_deg_kernel()(send2, recv2, zerosn)

    t0 = pl.pallas_call(
        _tc1_body,
        out_shape=jax.ShapeDtypeStruct((N_P, TW), _f32),
    )(x, W0, b0_2d, deg)

    s0p = _make_prop_kernel(TW)(t0, send3, recv3, zeros80)

    t1, aux = pl.pallas_call(
        _tc2_body,
        out_shape=[
            jax.ShapeDtypeStruct((N_P, H0), _f32),
            jax.ShapeDtypeStruct((N_P, 8), _f32),
        ],
    )(s0p, deg, b2_16)

    s1p = _make_prop_kernel(H0)(t1, send3, recv3, zeros64)

    t2 = pl.pallas_call(
        _tc3_body,
        grid=(NRB,),
        in_specs=[
            pl.BlockSpec((NC, RB, H0), lambda i: (0, i, 0)),
            pl.BlockSpec((RB, 8), lambda i: (i, 0)),
            pl.BlockSpec((H0, H1), lambda i: (0, 0)),
            pl.BlockSpec((8, H1), lambda i: (0, 0)),
            pl.BlockSpec((H1, 16), lambda i: (0, 0)),
        ],
        out_specs=pl.BlockSpec((RB, 16), lambda i: (i, 0)),
        out_shape=jax.ShapeDtypeStruct((N_P, 16), _f32),
    )(s1p, aux, W1, b1_2d, w2_16)

    pooled = _make_prop_pool_kernel()(
        t2, send3, recv3, zeros16, aux, batch_pad, iota128)

    return (pooled[0] + pooled[1])[:G, :D_OUT]


# speedup vs baseline: 18.4340x; 1.0145x over previous
import functools

import jax
import jax.numpy as jnp
from jax import lax
from jax.experimental import pallas as pl
from jax.experimental.pallas import tpu as pltpu
from jax.experimental.pallas import tpu_sc as plsc

N = 10000
E = 320000
D_IN = 128
H0 = 64
H1 = 128
D_OUT = 2
G = 100

NC = 2
NS = 16
NW = NC * NS
EPT = E // NW
CH = 125
NCH = EPT // CH
NB = 4
NGRP = NCH // NB
N_P = 10240
RPT = N_P // NS
ZB = 128

TW = 72
DW = 8

RB = 2000
NRB = N // RB

_f32 = jnp.float32


def _sc_mesh():
    return plsc.VectorSubcoreMesh(
        core_axis_name="c", subcore_axis_name="s", num_cores=NC, num_subcores=NS
    )


def _make_deg_kernel():
    @functools.partial(
        pl.kernel,
        out_type=jax.ShapeDtypeStruct((NC, 2, N_P), _f32),
        mesh=_sc_mesh(),
        compiler_params=pltpu.CompilerParams(
            use_tc_tiling_on_sc=False, needs_layout_passes=False),
        scratch_types=[
            pltpu.VMEM((EPT,), jnp.int32),
            pltpu.VMEM((EPT,), jnp.int32),
            pltpu.VMEM((N_P,), _f32),
            pltpu.VMEM((N_P,), _f32),
            pltpu.VMEM((NS, RPT), _f32),
            pltpu.VMEM((RPT,), _f32),
            pltpu.VMEM_SHARED((NS, N_P), _f32),
            pltpu.SemaphoreType.DMA,
        ],
    )
    def deg_kernel(send_hbm, recv_hbm, zerosn_hbm, deg_hbm,
                   sidx_v, ridx_v, degs_t, degr_t, tmp2, acc_v, stage, sem):
        cid = lax.axis_index("c")
        sid = lax.axis_index("s")
        wid = cid * NS + sid
        r0 = sid * RPT
        pltpu.sync_copy(send_hbm.at[wid], sidx_v)
        pltpu.sync_copy(recv_hbm.at[wid], ridx_v)
        pltpu.sync_copy(zerosn_hbm, degs_t)
        pltpu.sync_copy(zerosn_hbm, degr_t)
        ones16 = jnp.ones((16,), _f32)

        def count(k, _):
            base = k * 16
            plsc.addupdate_scatter(degs_t, [sidx_v[pl.ds(base, 16)]], ones16)
            plsc.addupdate_scatter(degr_t, [ridx_v[pl.ds(base, 16)]], ones16)
            return _

        lax.fori_loop(0, EPT // 16, count, None)

        for rnd, (tab, row) in enumerate(((degs_t, 0), (degr_t, 1))):
            if rnd:
                plsc.subcore_barrier()
            pltpu.sync_copy(tab, stage.at[sid])
            plsc.subcore_barrier()
            for k in range(NS):
                pltpu.async_copy(
                    stage.at[k, pl.ds(r0, RPT)], tmp2.at[k], sem)
            for k in range(NS):
                pltpu.make_async_copy(
                    stage.at[k, pl.ds(r0, RPT)], tmp2.at[k], sem).wait()
            for q in range(RPT // 16):
                s = pl.ds(q * 16, 16)
                v = tmp2[0, s]
                for k in range(1, NS):
                    v = v + tmp2[k, s]
                acc_v[s] = v
            pltpu.sync_copy(acc_v, deg_hbm.at[cid, row, pl.ds(r0, RPT)])

    return deg_kernel


def _make_prop_kernel(D):
    @functools.partial(
        pl.kernel,
        out_type=jax.ShapeDtypeStruct((NC, N_P, D), _f32),
        mesh=_sc_mesh(),
        compiler_params=pltpu.CompilerParams(use_tc_tiling_on_sc=False),
        scratch_types=[
            pltpu.VMEM((NCH, CH), jnp.int32),
            pltpu.VMEM((NCH, CH), jnp.int32),
            [pltpu.VMEM((CH, D), _f32) for _ in range(NB)],
            pltpu.VMEM((ZB, D), _f32),
            pltpu.VMEM_SHARED((N_P, D), _f32),
            [pltpu.SemaphoreType.DMA for _ in range(NB)],
        ],
    )
    def prop_kernel(table_hbm, send_hbm, recv_hbm, zeros_hbm, out_hbm,
                    sidx_v, ridx_v, rows, buf_v, acc, sems):
        cid = lax.axis_index("c")
        sid = lax.axis_index("s")
        wid = cid * NS + sid
        r0 = sid * RPT
        pltpu.sync_copy(zeros_hbm, buf_v)
        for z in range(RPT // ZB):
            pltpu.sync_copy(buf_v, acc.at[pl.ds(r0 + z * ZB, ZB)])
        pltpu.sync_copy(send_hbm.at[wid], sidx_v)
        pltpu.sync_copy(recv_hbm.at[wid], ridx_v)
        plsc.subcore_barrier()

        for b in range(NB):
            pltpu.async_copy(table_hbm.at[sidx_v.at[b]], rows[b], sems[b])

        def grp(g, _):
            for b in range(NB):
                j = g * NB + b
                pltpu.make_async_copy(
                    table_hbm.at[sidx_v.at[j]], rows[b], sems[b]).wait()
                pltpu.sync_copy(rows[b], acc.at[ridx_v.at[j]], add=True)
                pltpu.async_copy(
                    table_hbm.at[sidx_v.at[j + NB]], rows[b], sems[b])
            return _

        lax.fori_loop(0, NGRP - 1, grp, None)
        for b in range(NB):
            j = (NGRP - 1) * NB + b
            pltpu.make_async_copy(
                table_hbm.at[sidx_v.at[j]], rows[b], sems[b]).wait()
            pltpu.sync_copy(rows[b], acc.at[ridx_v.at[j]], add=True)
        plsc.subcore_barrier()
        pltpu.sync_copy(acc.at[pl.ds(r0, RPT)], out_hbm.at[cid, pl.ds(r0, RPT)])

    return prop_kernel


def _scales(d_ref):
    out_deg = d_ref[0, 0] + d_ref[1, 0]
    in_deg = d_ref[0, 1] + d_ref[1, 1]
    a = lax.rsqrt(jnp.maximum(out_deg, 1.0))
    c = lax.rsqrt(jnp.maximum(in_deg, 1.0))
    return a, c


def _tc1_body(x_ref, w0_ref, b0_ref, d_ref, t0_ref):
    a, _ = _scales(d_ref)
    h = jnp.dot(x_ref[...], w0_ref[...], preferred_element_type=_f32)
    h = h + b0_ref[0][None, :]
    h = jnp.concatenate([h, jnp.zeros((N_P - N, H0), _f32)], axis=0)
    t0_ref[...] = jnp.concatenate(
        [h * a[:, None], a[:, None], jnp.zeros((N_P, TW - H0 - 1), _f32)],
        axis=1)


def _tc2_body(s0_ref, d_ref, b2_ref, t1_ref, aux_ref):
    a, c = _scales(d_ref)
    s0 = s0_ref[0] + s0_ref[1]
    h0 = c[:, None] * s0[:, :H0]
    t1_ref[...] = a[:, None] * jnp.maximum(h0, 0.0)
    t = s0_ref[0, :, H0] + s0_ref[1, :, H0]
    ct = c * t
    b2m = b2_ref[...]
    aux_ref[...] = jnp.concatenate(
        [a[:, None], c[:, None], ct[:, None],
         (ct * b2m[0, 0])[:, None], (ct * b2m[0, 1])[:, None],
         jnp.zeros((N_P, 3), _f32)], axis=1)


def _tc3_body(s1_ref, aux_ref, w1_ref, b1_ref, w2_ref, t2_ref):
    s1 = s1_ref[0] + s1_ref[1]
    aux = aux_ref[...]
    a, c, ct = aux[:, 0], aux[:, 1], aux[:, 2]
    h1 = jnp.dot(c[:, None] * s1, w1_ref[...], preferred_element_type=_f32)
    h1 = h1 + ct[:, None] * b1_ref[0][None, :]
    r1 = jnp.maximum(h1, 0.0)
    t2_ref[...] = jnp.dot(a[:, None] * r1, w2_ref[...], preferred_element_type=_f32)


def _make_prop_pool_kernel():
    D = 16

    @functools.partial(
        pl.kernel,
        out_type=jax.ShapeDtypeStruct((NC, 128, 16), _f32),
        mesh=_sc_mesh(),
        compiler_params=pltpu.CompilerParams(
            use_tc_tiling_on_sc=False, needs_layout_passes=False),
        scratch_types=[
            pltpu.VMEM((NCH, CH), jnp.int32),
            pltpu.VMEM((NCH, CH), jnp.int32),
            [pltpu.VMEM((CH, D), _f32) for _ in range(NB)],
            pltpu.VMEM((ZB, D), _f32),
            pltpu.VMEM((RPT, D), _f32),
            pltpu.VMEM((RPT, 8), _f32),
            pltpu.VMEM((RPT,), jnp.int32),
            pltpu.VMEM((128,), jnp.int32),
            pltpu.VMEM((128, 16), _f32),
            pltpu.VMEM_SHARED((N_P, D), _f32),
            pltpu.VMEM_SHARED((128, 16), _f32),
            [pltpu.SemaphoreType.DMA for _ in range(NB)],
        ],
    )
    def prop_pool(table_hbm, send_hbm, recv_hbm, zeros_hbm,
                  aux_hbm, batch_hbm, iota_hbm, out_hbm,
                  sidx_v, ridx_v, rows, buf_v, s2_v, aux_v, b_v,
                  iota_v, pool_v, acc, pool_s, sems):
        cid = lax.axis_index("c")
        sid = lax.axis_index("s")
        wid = cid * NS + sid
        r0 = sid * RPT
        pltpu.sync_copy(zeros_hbm, buf_v)
        for z in range(RPT // ZB):
            pltpu.sync_copy(buf_v, acc.at[pl.ds(r0 + z * ZB, ZB)])
        pltpu.sync_copy(zeros_hbm, pool_v)

        @pl.when(sid == 0)
        def _():
            pltpu.sync_copy(pool_v, pool_s)

        pltpu.sync_copy(aux_hbm.at[pl.ds(r0, RPT)], aux_v)
        pltpu.sync_copy(batch_hbm.at[pl.ds(r0, RPT)], b_v)
        pltpu.sync_copy(iota_hbm, iota_v)
        pltpu.sync_copy(send_hbm.at[wid], sidx_v)
        pltpu.sync_copy(recv_hbm.at[wid], ridx_v)
        plsc.subcore_barrier()

        for b in range(NB):
            pltpu.async_copy(table_hbm.at[sidx_v.at[b]], rows[b], sems[b])

        def grp(g, _):
            for b in range(NB):
                j = g * NB + b
                pltpu.make_async_copy(
                    table_hbm.at[sidx_v.at[j]], rows[b], sems[b]).wait()
                pltpu.sync_copy(rows[b], acc.at[ridx_v.at[j]], add=True)
                pltpu.async_copy(
                    table_hbm.at[sidx_v.at[j + NB]], rows[b], sems[b])
            return _

        lax.fori_loop(0, NGRP - 1, grp, None)
        for b in range(NB):
            j = (NGRP - 1) * NB + b
            pltpu.make_async_copy(
                table_hbm.at[sidx_v.at[j]], rows[b], sems[b]).wait()
            pltpu.sync_copy(rows[b], acc.at[ridx_v.at[j]], add=True)
        plsc.subcore_barrier()

        pltpu.sync_copy(acc.at[pl.ds(r0, RPT)], s2_v)
        m = jnp.where(cid == 0, 1.0, 0.0).astype(_f32)
        lane = lax.iota(jnp.int32, 16)
        zcol = jnp.zeros((16,), jnp.int32)
        ocol = jnp.ones((16,), jnp.int32)

        def pool_body(g, _):
            base = g * 16
            rowi = base + lane
            s20 = plsc.load_gather(s2_v, [rowi, zcol])
            s21 = plsc.load_gather(s2_v, [rowi, ocol])
            cv = plsc.load_gather(aux_v, [rowi, ocol])
            p0 = plsc.load_gather(aux_v, [rowi, zcol + 3])
            p1 = plsc.load_gather(aux_v, [rowi, zcol + 4])
            bv = b_v[pl.ds(base, 16)]
            v0 = cv * s20 + m * p0
            v1 = cv * s21 + m * p1
            plsc.addupdate_scatter(pool_v, [bv, zcol], v0)
            plsc.addupdate_scatter(pool_v, [bv, ocol], v1)
            return _

        lax.fori_loop(0, RPT // 16, pool_body, None)
        pltpu.sync_copy(pool_v, pool_s.at[iota_v], add=True)
        plsc.subcore_barrier()

        @pl.when(sid == 0)
        def _():
            pltpu.sync_copy(pool_s, out_hbm.at[cid])

    return prop_pool


def _deg_spec():
    return pl.BlockSpec((NC, 2, N_P), lambda i: (0, 0, 0))


def kernel(x, senders, receivers, batch, num_graphs, W0, b0, W1, b1, W2, b2):
    send3 = senders.reshape(NW, NCH, CH)
    recv3 = receivers.reshape(NW, NCH, CH)
    send2 = senders.reshape(NW, EPT)
    recv2 = receivers.reshape(NW, EPT)
    b0_2d = jnp.broadcast_to(b0[None, :], (8, H0))
    b1_2d = jnp.broadcast_to(b1[None, :], (8, H1))
    b2_16 = jnp.zeros((8, 16), _f32).at[:, :D_OUT].set(b2[None, :])
    w2_16 = jnp.zeros((H1, 16), _f32).at[:, :D_OUT].set(W2)
    batch_pad = jnp.concatenate(
        [batch, jnp.full((N_P - N,), 127, batch.dtype)])
    iota128 = jnp.arange(128, dtype=jnp.int32)

    zerosn = jnp.zeros((N_P,), _f32)
    zeros16 = jnp.zeros((ZB, 16), _f32)
    zeros64 = jnp.zeros((ZB, H0), _f32)
    zeros80 = jnp.zeros((ZB, TW), _f32)

    deg = _make_deg_kernel()(send2, recv2, zerosn)

    t0 = pl.pallas_call(
        _tc1_body,
        out_shape=jax.ShapeDtypeStruct((N_P, TW), _f32),
    )(x, W0, b0_2d, deg)

    s0p = _make_prop_kernel(TW)(t0, send3, recv3, zeros80)

    t1, aux = pl.pallas_call(
        _tc2_body,
        out_shape=[
            jax.ShapeDtypeStruct((N_P, H0), _f32),
            jax.ShapeDtypeStruct((N_P, 8), _f32),
        ],
    )(s0p, deg, b2_16)

    s1p = _make_prop_kernel(H0)(t1, send3, recv3, zeros64)

    t2 = pl.pallas_call(
        _tc3_body,
        grid=(NRB,),
        in_specs=[
            pl.BlockSpec((NC, RB, H0), lambda i: (0, i, 0)),
            pl.BlockSpec((RB, 8), lambda i: (i, 0)),
            pl.BlockSpec((H0, H1), lambda i: (0, 0)),
            pl.BlockSpec((8, H1), lambda i: (0, 0)),
            pl.BlockSpec((H1, 16), lambda i: (0, 0)),
        ],
        out_specs=pl.BlockSpec((RB, 16), lambda i: (i, 0)),
        out_shape=jax.ShapeDtypeStruct((N_P, 16), _f32),
    )(s1p, aux, W1, b1_2d, w2_16)

    pooled = _make_prop_pool_kernel()(
        t2, send3, recv3, zeros16, aux, batch_pad, iota128)

    return (pooled[0] + pooled[1])[:G, :D_OUT]


# speedup vs baseline: 18.4503x; 1.0009x over previous
import functools

import jax
import jax.numpy as jnp
from jax import lax
from jax.experimental import pallas as pl
from jax.experimental.pallas import tpu as pltpu
from jax.experimental.pallas import tpu_sc as plsc

N = 10000
E = 320000
D_IN = 128
H0 = 64
H1 = 128
D_OUT = 2
G = 100

NC = 2
NS = 16
NW = NC * NS
EPT = E // NW
CH = 125
NCH = EPT // CH
NB = 4
NGRP = NCH // NB
N_P = 10240
RPT = N_P // NS
ZB = 128

TW = 72

RB = 2000
NRB = N // RB

_f32 = jnp.float32


def _sc_mesh():
    return plsc.VectorSubcoreMesh(
        core_axis_name="c", subcore_axis_name="s", num_cores=NC, num_subcores=NS
    )


def _make_deg_kernel():
    @functools.partial(
        pl.kernel,
        out_type=jax.ShapeDtypeStruct((NC, 2, N_P), _f32),
        mesh=_sc_mesh(),
        compiler_params=pltpu.CompilerParams(
            use_tc_tiling_on_sc=False, needs_layout_passes=False),
        scratch_types=[
            pltpu.VMEM((EPT,), jnp.int32),
            pltpu.VMEM((EPT,), jnp.int32),
            pltpu.VMEM((N_P,), _f32),
            pltpu.VMEM((N_P,), _f32),
            pltpu.VMEM((NS, RPT), _f32),
            pltpu.VMEM((RPT,), _f32),
            pltpu.VMEM_SHARED((NS, N_P), _f32),
            pltpu.SemaphoreType.DMA,
        ],
    )
    def deg_kernel(send_hbm, recv_hbm, zerosn_hbm, deg_hbm,
                   sidx_v, ridx_v, degs_t, degr_t, tmp2, acc_v, stage, sem):
        cid = lax.axis_index("c")
        sid = lax.axis_index("s")
        wid = cid * NS + sid
        r0 = sid * RPT
        pltpu.sync_copy(send_hbm.at[wid], sidx_v)
        pltpu.sync_copy(recv_hbm.at[wid], ridx_v)
        pltpu.sync_copy(zerosn_hbm, degs_t)
        pltpu.sync_copy(zerosn_hbm, degr_t)
        ones16 = jnp.ones((16,), _f32)

        def count(k, _):
            base = k * 16
            plsc.addupdate_scatter(degs_t, [sidx_v[pl.ds(base, 16)]], ones16)
            plsc.addupdate_scatter(degr_t, [ridx_v[pl.ds(base, 16)]], ones16)
            return _

        lax.fori_loop(0, EPT // 16, count, None)

        for rnd, (tab, row) in enumerate(((degs_t, 0), (degr_t, 1))):
            if rnd:
                plsc.subcore_barrier()
            pltpu.sync_copy(tab, stage.at[sid])
            plsc.subcore_barrier()
            for k in range(NS):
                pltpu.async_copy(
                    stage.at[k, pl.ds(r0, RPT)], tmp2.at[k], sem)
            for k in range(NS):
                pltpu.make_async_copy(
                    stage.at[k, pl.ds(r0, RPT)], tmp2.at[k], sem).wait()
            for q in range(RPT // 16):
                s = pl.ds(q * 16, 16)
                v = tmp2[0, s]
                for k in range(1, NS):
                    v = v + tmp2[k, s]
                acc_v[s] = v
            pltpu.sync_copy(acc_v, deg_hbm.at[cid, row, pl.ds(r0, RPT)])

    return deg_kernel


def _make_prop_kernel(D):
    @functools.partial(
        pl.kernel,
        out_type=jax.ShapeDtypeStruct((NC, N_P, D), _f32),
        mesh=_sc_mesh(),
        compiler_params=pltpu.CompilerParams(use_tc_tiling_on_sc=False),
        scratch_types=[
            pltpu.VMEM((NCH, CH), jnp.int32),
            pltpu.VMEM((NCH, CH), jnp.int32),
            [pltpu.VMEM((CH, D), _f32) for _ in range(NB)],
            pltpu.VMEM((ZB, D), _f32),
            pltpu.VMEM_SHARED((N_P, D), _f32),
            [pltpu.SemaphoreType.DMA for _ in range(NB)],
        ],
    )
    def prop_kernel(table_hbm, send_hbm, recv_hbm, zeros_hbm, out_hbm,
                    sidx_v, ridx_v, rows, buf_v, acc, sems):
        cid = lax.axis_index("c")
        sid = lax.axis_index("s")
        wid = cid * NS + sid
        r0 = sid * RPT
        pltpu.sync_copy(zeros_hbm, buf_v)
        for z in range(RPT // ZB):
            pltpu.sync_copy(buf_v, acc.at[pl.ds(r0 + z * ZB, ZB)])
        pltpu.sync_copy(send_hbm.at[wid], sidx_v)
        pltpu.sync_copy(recv_hbm.at[wid], ridx_v)
        plsc.subcore_barrier()

        for b in range(NB):
            pltpu.async_copy(table_hbm.at[sidx_v.at[b]], rows[b], sems[b])

        def grp(g, _):
            for b in range(NB):
                j = g * NB + b
                pltpu.make_async_copy(
                    table_hbm.at[sidx_v.at[j]], rows[b], sems[b]).wait()
                pltpu.sync_copy(rows[b], acc.at[ridx_v.at[j]], add=True)
                pltpu.async_copy(
                    table_hbm.at[sidx_v.at[j + NB]], rows[b], sems[b])
            return _

        lax.fori_loop(0, NGRP - 1, grp, None)
        for b in range(NB):
            j = (NGRP - 1) * NB + b
            pltpu.make_async_copy(
                table_hbm.at[sidx_v.at[j]], rows[b], sems[b]).wait()
            pltpu.sync_copy(rows[b], acc.at[ridx_v.at[j]], add=True)
        plsc.subcore_barrier()
        pltpu.sync_copy(acc.at[pl.ds(r0, RPT)], out_hbm.at[cid, pl.ds(r0, RPT)])

    return prop_kernel


def _scales(d_ref):
    out_deg = d_ref[0, 0] + d_ref[1, 0]
    in_deg = d_ref[0, 1] + d_ref[1, 1]
    a = lax.rsqrt(jnp.maximum(out_deg, 1.0))
    c = lax.rsqrt(jnp.maximum(in_deg, 1.0))
    return a, c


def _tc1_body(x_ref, w0_ref, b0_ref, d_ref, t0_ref):
    a, _ = _scales(d_ref)
    h = jnp.dot(x_ref[...], w0_ref[...], preferred_element_type=_f32)
    h = h + b0_ref[0][None, :]
    h = jnp.concatenate([h, jnp.zeros((N_P - N, H0), _f32)], axis=0)
    t0_ref[...] = jnp.concatenate(
        [h * a[:, None], a[:, None], jnp.zeros((N_P, TW - H0 - 1), _f32)],
        axis=1)


def _tc2_body(s0_ref, d_ref, b2_ref, t1_ref, aux_ref):
    a, c = _scales(d_ref)
    s0 = s0_ref[0] + s0_ref[1]
    h0 = c[:, None] * s0[:, :H0]
    t1_ref[...] = a[:, None] * jnp.maximum(h0, 0.0)
    t = s0_ref[0, :, H0] + s0_ref[1, :, H0]
    ct = c * t
    b2m = b2_ref[...]
    aux_ref[...] = jnp.concatenate(
        [a[:, None], c[:, None], ct[:, None],
         (ct * b2m[0, 0])[:, None], (ct * b2m[0, 1])[:, None],
         jnp.zeros((N_P, 3), _f32)], axis=1)


def _tc3_body(s1_ref, aux_ref, w1_ref, b1_ref, w2_ref, t2_ref):
    s1 = s1_ref[0] + s1_ref[1]
    aux = aux_ref[...]
    a, c, ct = aux[:, 0], aux[:, 1], aux[:, 2]
    h1 = jnp.dot(c[:, None] * s1, w1_ref[...], preferred_element_type=_f32)
    h1 = h1 + ct[:, None] * b1_ref[0][None, :]
    r1 = jnp.maximum(h1, 0.0)
    t2_ref[...] = jnp.dot(a[:, None] * r1, w2_ref[...], preferred_element_type=_f32)


def _make_prop_pool_kernel():
    D = 16

    @functools.partial(
        pl.kernel,
        out_type=jax.ShapeDtypeStruct((NC, 128, 16), _f32),
        mesh=_sc_mesh(),
        compiler_params=pltpu.CompilerParams(
            use_tc_tiling_on_sc=False, needs_layout_passes=False),
        scratch_types=[
            pltpu.VMEM((NCH, CH), jnp.int32),
            pltpu.VMEM((NCH, CH), jnp.int32),
            [pltpu.VMEM((CH, D), _f32) for _ in range(NB)],
            pltpu.VMEM((ZB, D), _f32),
            pltpu.VMEM((RPT, D), _f32),
            pltpu.VMEM((RPT, 8), _f32),
            pltpu.VMEM((RPT,), jnp.int32),
            pltpu.VMEM((128,), jnp.int32),
            pltpu.VMEM((128, 16), _f32),
            pltpu.VMEM_SHARED((N_P, D), _f32),
            pltpu.VMEM_SHARED((128, 16), _f32),
            [pltpu.SemaphoreType.DMA for _ in range(NB)],
        ],
    )
    def prop_pool(table_hbm, send_hbm, recv_hbm, zeros_hbm,
                  aux_hbm, batch_hbm, iota_hbm, out_hbm,
                  sidx_v, ridx_v, rows, buf_v, s2_v, aux_v, b_v,
                  iota_v, pool_v, acc, pool_s, sems):
        cid = lax.axis_index("c")
        sid = lax.axis_index("s")
        wid = cid * NS + sid
        r0 = sid * RPT
        pltpu.sync_copy(zeros_hbm, buf_v)
        for z in range(RPT // ZB):
            pltpu.sync_copy(buf_v, acc.at[pl.ds(r0 + z * ZB, ZB)])
        pltpu.sync_copy(zeros_hbm, pool_v)

        @pl.when(sid == 0)
        def _():
            pltpu.sync_copy(pool_v, pool_s)

        pltpu.sync_copy(aux_hbm.at[pl.ds(r0, RPT)], aux_v)
        pltpu.sync_copy(batch_hbm.at[pl.ds(r0, RPT)], b_v)
        pltpu.sync_copy(iota_hbm, iota_v)
        pltpu.sync_copy(send_hbm.at[wid], sidx_v)
        pltpu.sync_copy(recv_hbm.at[wid], ridx_v)
        plsc.subcore_barrier()

        for b in range(NB):
            pltpu.async_copy(table_hbm.at[sidx_v.at[b]], rows[b], sems[b])

        def grp(g, _):
            for b in range(NB):
                j = g * NB + b
                pltpu.make_async_copy(
                    table_hbm.at[sidx_v.at[j]], rows[b], sems[b]).wait()
                pltpu.sync_copy(rows[b], acc.at[ridx_v.at[j]], add=True)
                pltpu.async_copy(
                    table_hbm.at[sidx_v.at[j + NB]], rows[b], sems[b])
            return _

        lax.fori_loop(0, NGRP - 1, grp, None)
        for b in range(NB):
            j = (NGRP - 1) * NB + b
            pltpu.make_async_copy(
                table_hbm.at[sidx_v.at[j]], rows[b], sems[b]).wait()
            pltpu.sync_copy(rows[b], acc.at[ridx_v.at[j]], add=True)
        plsc.subcore_barrier()

        pltpu.sync_copy(acc.at[pl.ds(r0, RPT)], s2_v)
        m = jnp.where(cid == 0, 1.0, 0.0).astype(_f32)
        lane = lax.iota(jnp.int32, 16)
        zcol = jnp.zeros((16,), jnp.int32)
        ocol = jnp.ones((16,), jnp.int32)

        def pool_body(g, _):
            base = g * 16
            rowi = base + lane
            s20 = plsc.load_gather(s2_v, [rowi, zcol])
            s21 = plsc.load_gather(s2_v, [rowi, ocol])
            cv = plsc.load_gather(aux_v, [rowi, ocol])
            p0 = plsc.load_gather(aux_v, [rowi, zcol + 3])
            p1 = plsc.load_gather(aux_v, [rowi, zcol + 4])
            bv = b_v[pl.ds(base, 16)]
            v0 = cv * s20 + m * p0
            v1 = cv * s21 + m * p1
            plsc.addupdate_scatter(pool_v, [bv, zcol], v0)
            plsc.addupdate_scatter(pool_v, [bv, ocol], v1)
            return _

        lax.fori_loop(0, RPT // 16, pool_body, None)
        pltpu.sync_copy(pool_v, pool_s.at[iota_v], add=True)
        plsc.subcore_barrier()

        @pl.when(sid == 0)
        def _():
            pltpu.sync_copy(pool_s, out_hbm.at[cid])

    return prop_pool


def _deg_spec():
    return pl.BlockSpec((NC, 2, N_P), lambda i: (0, 0, 0))


def kernel(x, senders, receivers, batch, num_graphs, W0, b0, W1, b1, W2, b2):
    send3 = senders.reshape(NW, NCH, CH)
    recv3 = receivers.reshape(NW, NCH, CH)
    send2 = senders.reshape(NW, EPT)
    recv2 = receivers.reshape(NW, EPT)
    b0_2d = jnp.broadcast_to(b0[None, :], (8, H0))
    b1_2d = jnp.broadcast_to(b1[None, :], (8, H1))
    b2_16 = jnp.zeros((8, 16), _f32).at[:, :D_OUT].set(b2[None, :])
    w2_16 = jnp.zeros((H1, 16), _f32).at[:, :D_OUT].set(W2)
    batch_pad = jnp.concatenate(
        [batch, jnp.full((N_P - N,), 127, batch.dtype)])
    iota128 = jnp.arange(128, dtype=jnp.int32)

    zerosn = jnp.zeros((N_P,), _f32)
    zeros16 = jnp.zeros((ZB, 16), _f32)
    zeros64 = jnp.zeros((ZB, H0), _f32)
    zeros80 = jnp.zeros((ZB, TW), _f32)

    deg = _make_deg_kernel()(send2, recv2, zerosn)

    t0 = pl.pallas_call(
        _tc1_body,
        out_shape=jax.ShapeDtypeStruct((N_P, TW), _f32),
    )(x, W0, b0_2d, deg)

    s0p = _make_prop_kernel(TW)(t0, send3, recv3, zeros80)

    t1, aux = pl.pallas_call(
        _tc2_body,
        out_shape=[
            jax.ShapeDtypeStruct((N_P, H0), _f32),
            jax.ShapeDtypeStruct((N_P, 8), _f32),
        ],
    )(s0p, deg, b2_16)

    s1p = _make_prop_kernel(H0)(t1, send3, recv3, zeros64)

    t2 = pl.pallas_call(
        _tc3_body,
        grid=(NRB,),
        in_specs=[
            pl.BlockSpec((NC, RB, H0), lambda i: (0, i, 0)),
            pl.BlockSpec((RB, 8), lambda i: (i, 0)),
            pl.BlockSpec((H0, H1), lambda i: (0, 0)),
            pl.BlockSpec((8, H1), lambda i: (0, 0)),
            pl.BlockSpec((H1, 16), lambda i: (0, 0)),
        ],
        out_specs=pl.BlockSpec((RB, 16), lambda i: (i, 0)),
        out_shape=jax.ShapeDtypeStruct((N_P, 16), _f32),
    )(s1p, aux, W1, b1_2d, w2_16)

    pooled = _make_prop_pool_kernel()(
        t2, send3, recv3, zeros16, aux, batch_pad, iota128)

    return (pooled[0] + pooled[1])[:G, :D_OUT]
